# baseline (speedup 1.0000x reference)
import functools
import jax
import jax.numpy as jnp
from jax import lax
from jax.experimental import pallas as pl
from jax.experimental.pallas import tpu as pltpu

RADIUS = 0.0625
BR = 512


def _lrelu(x):
    return jnp.where(x >= 0, x, 0.1 * x)


def _pad_rows(x, mult):
    p = (-x.shape[0]) % mult
    if p:
        x = jnp.pad(x, ((0, p),) + ((0, 0),) * (x.ndim - 1))
    return x


def _mm_kernel(x_ref, w_ref, o_ref, *, act):
    y = jnp.dot(x_ref[...], w_ref[...], preferred_element_type=jnp.float32)
    o_ref[...] = _lrelu(y) if act else y


def _mm(x, W, act=True):
    M, Ci = x.shape
    Co = W.shape[1]
    xp = _pad_rows(x, BR)
    Mp = xp.shape[0]
    out = pl.pallas_call(
        functools.partial(_mm_kernel, act=act),
        grid=(Mp // BR,),
        in_specs=[pl.BlockSpec((BR, Ci), lambda i: (i, 0)),
                  pl.BlockSpec((Ci, Co), lambda i: (0, 0))],
        out_specs=pl.BlockSpec((BR, Co), lambda i: (i, 0)),
        out_shape=jax.ShapeDtypeStruct((Mp, Co), jnp.float32),
    )(xp, W)
    return out[:M]


def _tail_kernel(x_ref, s_ref, wb_ref, wc_ref, ws_ref, o_ref):
    h = _lrelu(jnp.dot(x_ref[...], wb_ref[...], preferred_element_type=jnp.float32))
    y = jnp.dot(h, wc_ref[...], preferred_element_type=jnp.float32)
    sh = s_ref[...]
    if ws_ref is not None:
        sh = jnp.dot(sh, ws_ref[...], preferred_element_type=jnp.float32)
    o_ref[...] = _lrelu(y + sh)


def _tail_kernel_nows(x_ref, s_ref, wb_ref, wc_ref, o_ref):
    _tail_kernel(x_ref, s_ref, wb_ref, wc_ref, None, o_ref)


def _tail(x, short, Wb, Wc, Ws=None):
    M, m = x.shape
    o = Wc.shape[1]
    si = short.shape[1]
    xp = _pad_rows(x, BR)
    sp = _pad_rows(short, BR)
    Mp = xp.shape[0]
    specs = [pl.BlockSpec((BR, m), lambda i: (i, 0)),
             pl.BlockSpec((BR, si), lambda i: (i, 0)),
             pl.BlockSpec((m, m), lambda i: (0, 0)),
             pl.BlockSpec((m, o), lambda i: (0, 0))]
    args = [xp, sp, Wb, Wc]
    if Ws is not None:
        specs.append(pl.BlockSpec((si, o), lambda i: (0, 0)))
        args.append(Ws)
        body = _tail_kernel
    else:
        body = _tail_kernel_nows
    out = pl.pallas_call(
        body,
        grid=(Mp // BR,),
        in_specs=specs,
        out_specs=pl.BlockSpec((BR, o), lambda i: (i, 0)),
        out_shape=jax.ShapeDtypeStruct((Mp, o), jnp.float32),
    )(*args)
    return out[:M]


def _dec_kernel(g_ref, f_ref, w1_ref, w2_ref, o_ref, *, act):
    y = jnp.dot(g_ref[...], w1_ref[...], preferred_element_type=jnp.float32)
    y += jnp.dot(f_ref[...], w2_ref[...], preferred_element_type=jnp.float32)
    o_ref[...] = _lrelu(y) if act else y


def _dec(g, f, W, act):
    M, cg = g.shape
    cf = f.shape[1]
    co = W.shape[1]
    W1, W2 = W[:cg], W[cg:]
    gp = _pad_rows(g, BR)
    fp = _pad_rows(f, BR)
    Mp = gp.shape[0]
    out = pl.pallas_call(
        functools.partial(_dec_kernel, act=act),
        grid=(Mp // BR,),
        in_specs=[pl.BlockSpec((BR, cg), lambda i: (i, 0)),
                  pl.BlockSpec((BR, cf), lambda i: (i, 0)),
                  pl.BlockSpec((cg, co), lambda i: (0, 0)),
                  pl.BlockSpec((cf, co), lambda i: (0, 0))],
        out_specs=pl.BlockSpec((BR, co), lambda i: (i, 0)),
        out_shape=jax.ShapeDtypeStruct((Mp, co), jnp.float32),
    )(gp, fp, W1, W2)
    return out[:M]


def _weights(q_points, s_points, neighbors):
    n_pts = s_points[neighbors]
    d2 = jnp.sum((q_points[:, None, :] - n_pts) ** 2, axis=-1)
    return 1.0 / (1.0 + d2 / (RADIUS * RADIUS))


def _wpool(feats, w, neighbors):
    g = feats[neighbors] * w[..., None]
    return jnp.sum(g, axis=1) / neighbors.shape[1]


def _maxpool(feats, neighbors):
    return jnp.max(feats[neighbors], axis=1)


def kernel(feats, points0, points1, points2, points3,
           neighbors0, neighbors1, neighbors2, neighbors3,
           subsampling0, subsampling1, subsampling2,
           upsampling1, upsampling2, params):
    p = params
    w_n0 = _weights(points0, points0, neighbors0)
    w_s0 = _weights(points1, points0, subsampling0)
    w_n1 = _weights(points1, points1, neighbors1)
    w_s1 = _weights(points2, points1, subsampling1)
    w_n2 = _weights(points2, points2, neighbors2)
    w_s2 = _weights(points3, points2, subsampling2)
    w_n3 = _weights(points3, points3, neighbors3)

    wp = _wpool(feats, w_n0, neighbors0)
    wp8 = jnp.pad(wp, ((0, 0), (0, 7)))
    W8 = jnp.pad(p["e11"]["W"], ((0, 7), (0, 0)))
    f1 = _mm(wp8, W8, act=True)

    def resnet(f, w, nb, pr, strided):
        x = _mm(f, pr["Wa"], act=True)
        x = _wpool(x, w, nb)
        short = _maxpool(f, nb) if strided else f
        return _tail(x, short, pr["Wb"], pr["Wc"], pr.get("Ws"))

    f1 = resnet(f1, w_n0, neighbors0, p["e12"], False)
    f2 = resnet(f1, w_s0, subsampling0, p["e21"], True)
    f2 = resnet(f2, w_n1, neighbors1, p["e22"], False)
    f2 = resnet(f2, w_n1, neighbors1, p["e23"], False)
    f3 = resnet(f2, w_s1, subsampling1, p["e31"], True)
    f3 = resnet(f3, w_n2, neighbors2, p["e32"], False)
    f3 = resnet(f3, w_n2, neighbors2, p["e33"], False)
    f4 = resnet(f3, w_s2, subsampling2, p["e41"], True)
    f4 = resnet(f4, w_n3, neighbors3, p["e42"], False)
    f4 = resnet(f4, w_n3, neighbors3, p["e43"], False)

    g3 = f4[upsampling2[:, 0]]
    l3 = _dec(g3, f3, p["d3"]["W"], act=True)
    g2 = l3[upsampling1[:, 0]]
    l2 = _dec(g2, f2, p["d2"]["W"], act=False)
    return (l2, l3, f4)


# speedup vs baseline: 2.4244x; 2.4244x over previous
import functools
import jax
import jax.numpy as jnp
from jax import lax
from jax.experimental import pallas as pl
from jax.experimental.pallas import tpu as pltpu
from jax.experimental.pallas import tpu_sc as plsc

RADIUS = 0.0625
KNN = 32
BR = 512
NC, NS = 2, 16
NW = NC * NS
ROW_ALIGN = 256

_CP = pltpu.CompilerParams(needs_layout_passes=False)


def _lrelu(x):
    return jnp.where(x >= 0, x, 0.1 * x)


def _pad_rows(x, mult):
    p = (-x.shape[0]) % mult
    if p:
        x = jnp.pad(x, ((0, p),) + ((0, 0),) * (x.ndim - 1))
    return x


def _padM(M):
    return -(-M // ROW_ALIGN) * ROW_ALIGN


def _mm_kernel(x_ref, w_ref, o_ref, *, act):
    y = jnp.dot(x_ref[...], w_ref[...], preferred_element_type=jnp.float32)
    o_ref[...] = _lrelu(y) if act else y


def _mm(x, W, act=True):
    M, Ci = x.shape
    Co = W.shape[1]
    xp = _pad_rows(x, BR)
    Mp = xp.shape[0]
    out = pl.pallas_call(
        functools.partial(_mm_kernel, act=act),
        grid=(Mp // BR,),
        in_specs=[pl.BlockSpec((BR, Ci), lambda i: (i, 0)),
                  pl.BlockSpec((Ci, Co), lambda i: (0, 0))],
        out_specs=pl.BlockSpec((BR, Co), lambda i: (i, 0)),
        out_shape=jax.ShapeDtypeStruct((Mp, Co), jnp.float32),
    )(xp, W)
    return out[:M]


def _tail_kernel(x_ref, s_ref, wb_ref, wc_ref, ws_ref, o_ref):
    h = _lrelu(jnp.dot(x_ref[...], wb_ref[...], preferred_element_type=jnp.float32))
    y = jnp.dot(h, wc_ref[...], preferred_element_type=jnp.float32)
    sh = s_ref[...]
    if ws_ref is not None:
        sh = jnp.dot(sh, ws_ref[...], preferred_element_type=jnp.float32)
    o_ref[...] = _lrelu(y + sh)


def _tail_kernel_nows(x_ref, s_ref, wb_ref, wc_ref, o_ref):
    _tail_kernel(x_ref, s_ref, wb_ref, wc_ref, None, o_ref)


def _tail(x, short, Wb, Wc, Ws=None):
    M, m = x.shape
    o = Wc.shape[1]
    si = short.shape[1]
    xp = _pad_rows(x, BR)
    sp = _pad_rows(short, BR)
    Mp = xp.shape[0]
    specs = [pl.BlockSpec((BR, m), lambda i: (i, 0)),
             pl.BlockSpec((BR, si), lambda i: (i, 0)),
             pl.BlockSpec((m, m), lambda i: (0, 0)),
             pl.BlockSpec((m, o), lambda i: (0, 0))]
    args = [xp, sp, Wb, Wc]
    if Ws is not None:
        specs.append(pl.BlockSpec((si, o), lambda i: (0, 0)))
        args.append(Ws)
        body = _tail_kernel
    else:
        body = _tail_kernel_nows
    out = pl.pallas_call(
        body,
        grid=(Mp // BR,),
        in_specs=specs,
        out_specs=pl.BlockSpec((BR, o), lambda i: (i, 0)),
        out_shape=jax.ShapeDtypeStruct((Mp, o), jnp.float32),
    )(*args)
    return out[:M]


def _dec_kernel(g_ref, f_ref, w1_ref, w2_ref, o_ref, *, act):
    y = jnp.dot(g_ref[...], w1_ref[...], preferred_element_type=jnp.float32)
    y += jnp.dot(f_ref[...], w2_ref[...], preferred_element_type=jnp.float32)
    o_ref[...] = _lrelu(y) if act else y


def _dec(g, f, W, act):
    M, cg = g.shape
    cf = f.shape[1]
    co = W.shape[1]
    W1, W2 = W[:cg], W[cg:]
    gp = _pad_rows(g, BR)
    fp = _pad_rows(f, BR)
    Mp = gp.shape[0]
    out = pl.pallas_call(
        functools.partial(_dec_kernel, act=act),
        grid=(Mp // BR,),
        in_specs=[pl.BlockSpec((BR, cg), lambda i: (i, 0)),
                  pl.BlockSpec((BR, cf), lambda i: (i, 0)),
                  pl.BlockSpec((cg, co), lambda i: (0, 0)),
                  pl.BlockSpec((cf, co), lambda i: (0, 0))],
        out_specs=pl.BlockSpec((BR, co), lambda i: (i, 0)),
        out_shape=jax.ShapeDtypeStruct((Mp, co), jnp.float32),
    )(gp, fp, W1, W2)
    return out[:M]


def _sc_mesh():
    return plsc.VectorSubcoreMesh(core_axis_name="c", subcore_axis_name="s")


def _wid():
    return lax.axis_index("s") * NC + lax.axis_index("c")


def _full(v):
    return jnp.full((16,), v, jnp.int32)


def _sc_weights(qpad, spad, nb_flat):
    Mp = qpad.shape[0]
    qflat = qpad.reshape(-1)
    QB = 8
    G = QB * KNN
    rows_pw = Mp // NW
    nchunk = rows_pw // QB
    inv_r2 = 1.0 / (RADIUS * RADIUS)

    @functools.partial(
        pl.kernel,
        out_type=jax.ShapeDtypeStruct((Mp * KNN,), jnp.float32),
        mesh=_sc_mesh(),
        compiler_params=_CP,
        scratch_types=[
            pltpu.VMEM((128,), jnp.int32),
            pltpu.VMEM((128,), jnp.int32),
            pltpu.VMEM((QB * 16,), jnp.float32),
            pltpu.VMEM((128, 128), jnp.float32),
            pltpu.VMEM((128, 128), jnp.float32),
            pltpu.VMEM((KNN * 16,), jnp.float32),
            pltpu.VMEM((G,), jnp.float32),
            pltpu.SemaphoreType.DMA,
        ])
    def body(qp_h, sp_h, nb_h, w_h, idx0_v, idx1_v, qp_v, rows0_v,
             rows1_v, ev_v, wout_v, sem):
        base = _wid() * rows_pw
        kiota = lax.broadcasted_iota(jnp.int32, (16,), 0)
        zero16 = jnp.zeros((16,), jnp.float32)
        idx_bufs = [idx0_v, idx1_v]
        row_bufs = [rows0_v, rows1_v]

        def chunk(ci, carry):
            q0 = base + ci * QB
            pltpu.sync_copy(qp_h.at[pl.ds(q0 * 16, QB * 16)], qp_v)
            for g in range(2):
                pltpu.sync_copy(nb_h.at[pl.ds(q0 * KNN + g * 128, 128)],
                                idx_bufs[g])
                pltpu.async_copy(sp_h.at[idx_bufs[g]], row_bufs[g], sem).wait()
            for q in range(QB):
                qx = plsc.load_gather(qp_v, [_full(q * 16 + 0)])
                qy = plsc.load_gather(qp_v, [_full(q * 16 + 1)])
                qz = plsc.load_gather(qp_v, [_full(q * 16 + 2)])
                qvec = jnp.where(kiota == 0, qx,
                                 jnp.where(kiota == 1, qy,
                                           jnp.where(kiota == 2, qz, zero16)))
                buf = row_bufs[q // 4]
                for j in range(KNN):
                    v = plsc.load_gather(buf, [_full((q % 4) * KNN + j), kiota])
                    d = v - qvec
                    ev_v[pl.ds(j * 16, 16)] = d * d
                for g2 in range(KNN // 16):
                    eb = g2 * 256
                    d2 = (plsc.load_gather(ev_v, [kiota * 16 + eb])
                          + plsc.load_gather(ev_v, [kiota * 16 + eb + 1])
                          + plsc.load_gather(ev_v, [kiota * 16 + eb + 2]))
                    wout_v[pl.ds(q * KNN + g2 * 16, 16)] = 1.0 / (1.0 + d2 * inv_r2)
            pltpu.sync_copy(wout_v, w_h.at[pl.ds(q0 * KNN, G)])
            return carry

        lax.fori_loop(0, nchunk, chunk, 0)

    return body(qflat, spad, nb_flat)


_WPOOL_QB = {16: 4, 32: 4, 64: 4, 128: 4, 256: 2}


def _sc_wpool(feats, nb_flat, w_flat, C):
    CP = feats.shape[1]
    Mp = w_flat.shape[0] // KNN
    QB = _WPOOL_QB[C]
    G = QB * KNN
    rows_pw = Mp // NW
    nchunk = rows_pw // QB
    scale = 1.0 / KNN

    @functools.partial(
        pl.kernel,
        out_type=jax.ShapeDtypeStruct((Mp * C,), jnp.float32),
        mesh=_sc_mesh(),
        compiler_params=_CP,
        scratch_types=[
            pltpu.VMEM((G,), jnp.int32),
            pltpu.VMEM((G,), jnp.float32),
            pltpu.VMEM((G, CP), jnp.float32),
            pltpu.VMEM((QB * C,), jnp.float32),
            pltpu.SemaphoreType.DMA,
        ])
    def body(f_h, nb_h, w_h, out_h, idx_v, w_v, rows_v, out_v, sem):
        base = _wid() * rows_pw
        kiota = lax.broadcasted_iota(jnp.int32, (16,), 0)

        def chunk(ci, carry):
            q0 = base + ci * QB
            pltpu.sync_copy(nb_h.at[pl.ds(q0 * KNN, G)], idx_v)
            pltpu.sync_copy(w_h.at[pl.ds(q0 * KNN, G)], w_v)
            pltpu.async_copy(f_h.at[idx_v], rows_v, sem).wait()
            for q in range(QB):
                accs = [jnp.zeros((16,), jnp.float32) for _ in range(C // 16)]
                for k in range(KNN):
                    wk = plsc.load_gather(w_v, [_full(q * KNN + k)])
                    row = q * KNN + k
                    for cc in range(C // 16):
                        rv = plsc.load_gather(rows_v, [_full(row), kiota + cc * 16])
                        accs[cc] = accs[cc] + wk * rv
                for cc in range(C // 16):
                    out_v[pl.ds(q * C + cc * 16, 16)] = accs[cc] * scale
            pltpu.sync_copy(out_v, out_h.at[pl.ds(q0 * C, QB * C)])
            return carry

        lax.fori_loop(0, nchunk, chunk, 0)

    return body(feats, nb_flat, w_flat).reshape(Mp, C)


def _wpool_any(feats, nb_flat, w_flat):
    Ns, C = feats.shape
    if C < 128:
        feats = jnp.pad(feats, ((0, 0), (0, 128 - C)))
    return _sc_wpool(feats, nb_flat, w_flat, C)


_MAXP_QB = {128: 4, 256: 2, 512: 1}


def _sc_maxpool(feats, nb_flat, Mp):
    C = feats.shape[1]
    QB = _MAXP_QB[C]
    G = QB * KNN
    rows_pw = Mp // NW
    nchunk = rows_pw // QB

    @functools.partial(
        pl.kernel,
        out_type=jax.ShapeDtypeStruct((Mp * C,), jnp.float32),
        mesh=_sc_mesh(),
        scratch_types=[
            pltpu.VMEM((G,), jnp.int32),
            pltpu.VMEM((G, C), jnp.float32),
            pltpu.VMEM((QB * C,), jnp.float32),
            pltpu.SemaphoreType.DMA,
        ])
    def body(f_h, nb_h, out_h, idx_v, rows_v, out_v, sem):
        base = _wid() * rows_pw

        def chunk(ci, carry):
            q0 = base + ci * QB
            pltpu.sync_copy(nb_h.at[pl.ds(q0 * KNN, G)], idx_v)
            pltpu.async_copy(f_h.at[idx_v], rows_v, sem).wait()
            for q in range(QB):
                for cc in range(C // 16):
                    acc = rows_v[q * KNN, pl.ds(cc * 16, 16)]
                    for k in range(1, KNN):
                        acc = jnp.maximum(acc, rows_v[q * KNN + k, pl.ds(cc * 16, 16)])
                    out_v[pl.ds(q * C + cc * 16, 16)] = acc
            pltpu.sync_copy(out_v, out_h.at[pl.ds(q0 * C, QB * C)])
            return carry

        lax.fori_loop(0, nchunk, chunk, 0)

    return body(feats, nb_flat).reshape(Mp, C)


def _sc_rowgather(table, idx_flat):
    C = table.shape[1]
    Mp = idx_flat.shape[0]
    QB = 8
    rows_pw = Mp // NW
    nchunk = rows_pw // QB

    @functools.partial(
        pl.kernel,
        out_type=jax.ShapeDtypeStruct((Mp, C), jnp.float32),
        mesh=_sc_mesh(),
        scratch_types=[
            pltpu.VMEM((QB,), jnp.int32),
            pltpu.VMEM((QB, C), jnp.float32),
            pltpu.SemaphoreType.DMA,
        ])
    def body(t_h, i_h, out_h, idx_v, rows_v, sem):
        base = _wid() * rows_pw

        def chunk(ci, carry):
            q0 = base + ci * QB
            pltpu.sync_copy(i_h.at[pl.ds(q0, QB)], idx_v)
            pltpu.async_copy(t_h.at[idx_v], rows_v, sem).wait()
            pltpu.sync_copy(rows_v, out_h.at[pl.ds(q0, QB)])
            return carry

        lax.fori_loop(0, nchunk, chunk, 0)

    return body(table, idx_flat)


def _lanepad16(x):
    return jnp.pad(x, ((0, 0), (0, 16 - x.shape[1])))


def kernel(feats, points0, points1, points2, points3,
           neighbors0, neighbors1, neighbors2, neighbors3,
           subsampling0, subsampling1, subsampling2,
           upsampling1, upsampling2, params):
    p = params
    N1, N2, N3, N4 = points0.shape[0], points1.shape[0], points2.shape[0], points3.shape[0]
    Mp1, Mp2, Mp3, Mp4 = _padM(N1), _padM(N2), _padM(N3), _padM(N4)

    nb0 = _pad_rows(neighbors0, ROW_ALIGN).reshape(-1)
    nb1 = _pad_rows(neighbors1, ROW_ALIGN).reshape(-1)
    nb2 = _pad_rows(neighbors2, ROW_ALIGN).reshape(-1)
    nb3 = _pad_rows(neighbors3, ROW_ALIGN).reshape(-1)
    ss0 = _pad_rows(subsampling0, ROW_ALIGN).reshape(-1)
    ss1 = _pad_rows(subsampling1, ROW_ALIGN).reshape(-1)
    ss2 = _pad_rows(subsampling2, ROW_ALIGN).reshape(-1)

    sp0, sp1, sp2, sp3 = (jnp.pad(x, ((0, 0), (0, 125)))
                          for x in (points0, points1, points2, points3))
    qp0, qp1, qp2, qp3 = (_pad_rows(_lanepad16(x), ROW_ALIGN)
                          for x in (points0, points1, points2, points3))

    w_n0 = _sc_weights(qp0, sp0, nb0)
    w_s0 = _sc_weights(qp1, sp0, ss0)
    w_n1 = _sc_weights(qp1, sp1, nb1)
    w_s1 = _sc_weights(qp2, sp1, ss1)
    w_n2 = _sc_weights(qp2, sp2, nb2)
    w_s2 = _sc_weights(qp3, sp2, ss2)
    w_n3 = _sc_weights(qp3, sp3, nb3)

    wp = _wpool_any(_lanepad16(feats), nb0, w_n0)[:N1, :8]
    W8 = jnp.pad(p["e11"]["W"], ((0, 7), (0, 0)))
    f1 = _mm(wp, W8, act=True)

    def resnet(f, Mq, wtab, nbf, Mp, pr, strided):
        x = _mm(f, pr["Wa"], act=True)
        xp = _wpool_any(x, nbf, wtab)[:Mq]
        short = _sc_maxpool(f, nbf, Mp)[:Mq] if strided else f
        return _tail(xp, short, pr["Wb"], pr["Wc"], pr.get("Ws"))

    f1 = resnet(f1, N1, w_n0, nb0, Mp1, p["e12"], False)
    f2 = resnet(f1, N2, w_s0, ss0, Mp2, p["e21"], True)
    f2 = resnet(f2, N2, w_n1, nb1, Mp2, p["e22"], False)
    f2 = resnet(f2, N2, w_n1, nb1, Mp2, p["e23"], False)
    f3 = resnet(f2, N3, w_s1, ss1, Mp3, p["e31"], True)
    f3 = resnet(f3, N3, w_n2, nb2, Mp3, p["e32"], False)
    f3 = resnet(f3, N3, w_n2, nb2, Mp3, p["e33"], False)
    f4 = resnet(f3, N4, w_s2, ss2, Mp4, p["e41"], True)
    f4 = resnet(f4, N4, w_n3, nb3, Mp4, p["e42"], False)
    f4 = resnet(f4, N4, w_n3, nb3, Mp4, p["e43"], False)

    up2 = _pad_rows(upsampling2[:, 0], ROW_ALIGN)
    up1 = _pad_rows(upsampling1[:, 0], ROW_ALIGN)
    g3 = _sc_rowgather(f4, up2)[:N3]
    l3 = _dec(g3, f3, p["d3"]["W"], act=True)
    g2 = _sc_rowgather(l3, up1)[:N2]
    l2 = _dec(g2, f2, p["d2"]["W"], act=False)
    return (l2, l3, f4)


# speedup vs baseline: 2.6292x; 1.0844x over previous
import functools
import jax
import jax.numpy as jnp
from jax import lax
from jax.experimental import pallas as pl
from jax.experimental.pallas import tpu as pltpu
from jax.experimental.pallas import tpu_sc as plsc

RADIUS = 0.0625
KNN = 32
BR = 512
NC, NS = 2, 16
NW = NC * NS
ROW_ALIGN = 256

_CP = pltpu.CompilerParams(needs_layout_passes=False)


def _lrelu(x):
    return jnp.where(x >= 0, x, 0.1 * x)


def _pad_rows(x, mult):
    p = (-x.shape[0]) % mult
    if p:
        x = jnp.pad(x, ((0, p),) + ((0, 0),) * (x.ndim - 1))
    return x


def _padM(M):
    return -(-M // ROW_ALIGN) * ROW_ALIGN


def _mm_kernel(x_ref, w_ref, o_ref, *, act):
    y = jnp.dot(x_ref[...], w_ref[...], preferred_element_type=jnp.float32)
    o_ref[...] = _lrelu(y) if act else y


def _mm(x, W, act=True):
    M, Ci = x.shape
    Co = W.shape[1]
    xp = _pad_rows(x, BR)
    Mp = xp.shape[0]
    out = pl.pallas_call(
        functools.partial(_mm_kernel, act=act),
        grid=(Mp // BR,),
        in_specs=[pl.BlockSpec((BR, Ci), lambda i: (i, 0)),
                  pl.BlockSpec((Ci, Co), lambda i: (0, 0))],
        out_specs=pl.BlockSpec((BR, Co), lambda i: (i, 0)),
        out_shape=jax.ShapeDtypeStruct((Mp, Co), jnp.float32),
    )(xp, W)
    return out[:M]


def _tail_kernel(x_ref, s_ref, wb_ref, wc_ref, ws_ref, o_ref):
    h = _lrelu(jnp.dot(x_ref[...], wb_ref[...], preferred_element_type=jnp.float32))
    y = jnp.dot(h, wc_ref[...], preferred_element_type=jnp.float32)
    sh = s_ref[...]
    if ws_ref is not None:
        sh = jnp.dot(sh, ws_ref[...], preferred_element_type=jnp.float32)
    o_ref[...] = _lrelu(y + sh)


def _tail_kernel_nows(x_ref, s_ref, wb_ref, wc_ref, o_ref):
    _tail_kernel(x_ref, s_ref, wb_ref, wc_ref, None, o_ref)


def _tail(x, short, Wb, Wc, Ws=None):
    M, m = x.shape
    o = Wc.shape[1]
    si = short.shape[1]
    xp = _pad_rows(x, BR)
    sp = _pad_rows(short, BR)
    Mp = xp.shape[0]
    specs = [pl.BlockSpec((BR, m), lambda i: (i, 0)),
             pl.BlockSpec((BR, si), lambda i: (i, 0)),
             pl.BlockSpec((m, m), lambda i: (0, 0)),
             pl.BlockSpec((m, o), lambda i: (0, 0))]
    args = [xp, sp, Wb, Wc]
    if Ws is not None:
        specs.append(pl.BlockSpec((si, o), lambda i: (0, 0)))
        args.append(Ws)
        body = _tail_kernel
    else:
        body = _tail_kernel_nows
    out = pl.pallas_call(
        body,
        grid=(Mp // BR,),
        in_specs=specs,
        out_specs=pl.BlockSpec((BR, o), lambda i: (i, 0)),
        out_shape=jax.ShapeDtypeStruct((Mp, o), jnp.float32),
    )(*args)
    return out[:M]


def _dec_kernel(g_ref, f_ref, w1_ref, w2_ref, o_ref, *, act):
    y = jnp.dot(g_ref[...], w1_ref[...], preferred_element_type=jnp.float32)
    y += jnp.dot(f_ref[...], w2_ref[...], preferred_element_type=jnp.float32)
    o_ref[...] = _lrelu(y) if act else y


def _dec(g, f, W, act):
    M, cg = g.shape
    cf = f.shape[1]
    co = W.shape[1]
    W1, W2 = W[:cg], W[cg:]
    gp = _pad_rows(g, BR)
    fp = _pad_rows(f, BR)
    Mp = gp.shape[0]
    out = pl.pallas_call(
        functools.partial(_dec_kernel, act=act),
        grid=(Mp // BR,),
        in_specs=[pl.BlockSpec((BR, cg), lambda i: (i, 0)),
                  pl.BlockSpec((BR, cf), lambda i: (i, 0)),
                  pl.BlockSpec((cg, co), lambda i: (0, 0)),
                  pl.BlockSpec((cf, co), lambda i: (0, 0))],
        out_specs=pl.BlockSpec((BR, co), lambda i: (i, 0)),
        out_shape=jax.ShapeDtypeStruct((Mp, co), jnp.float32),
    )(gp, fp, W1, W2)
    return out[:M]


def _sc_mesh():
    return plsc.VectorSubcoreMesh(core_axis_name="c", subcore_axis_name="s")


def _wid():
    return lax.axis_index("s") * NC + lax.axis_index("c")


def _full(v):
    return jnp.full((16,), v, jnp.int32)


def _sc_weights(qpad, spad, nb_flat):
    Mp = qpad.shape[0]
    qflat = qpad.reshape(-1)
    QB = 8
    G = QB * KNN
    rows_pw = Mp // NW
    nchunk = rows_pw // QB
    inv_r2 = 1.0 / (RADIUS * RADIUS)

    @functools.partial(
        pl.kernel,
        out_type=jax.ShapeDtypeStruct((Mp * KNN,), jnp.float32),
        mesh=_sc_mesh(),
        compiler_params=_CP,
        scratch_types=[
            pltpu.VMEM((128,), jnp.int32),
            pltpu.VMEM((128,), jnp.int32),
            pltpu.VMEM((QB * 16,), jnp.float32),
            pltpu.VMEM((128, 128), jnp.float32),
            pltpu.VMEM((128, 128), jnp.float32),
            pltpu.VMEM((KNN * 16,), jnp.float32),
            pltpu.VMEM((G,), jnp.float32),
            pltpu.SemaphoreType.DMA,
            pltpu.SemaphoreType.DMA,
        ])
    def body(qp_h, sp_h, nb_h, w_h, idx0_v, idx1_v, qp_v, rows0_v,
             rows1_v, ev_v, wout_v, sem0, sem1):
        base = _wid() * rows_pw
        kiota = lax.broadcasted_iota(jnp.int32, (16,), 0)
        zero16 = jnp.zeros((16,), jnp.float32)
        idx_bufs = [idx0_v, idx1_v]
        row_bufs = [rows0_v, rows1_v]
        sems = [sem0, sem1]

        def compute(q, buf, q0):
            qx = plsc.load_gather(qp_v, [_full(q * 16 + 0)])
            qy = plsc.load_gather(qp_v, [_full(q * 16 + 1)])
            qz = plsc.load_gather(qp_v, [_full(q * 16 + 2)])
            qvec = jnp.where(kiota == 0, qx,
                             jnp.where(kiota == 1, qy,
                                       jnp.where(kiota == 2, qz, zero16)))
            for j in range(KNN):
                v = plsc.load_gather(buf, [_full((q % 4) * KNN + j), kiota])
                d = v - qvec
                ev_v[pl.ds(j * 16, 16)] = d * d
            for g2 in range(KNN // 16):
                eb = g2 * 256
                d2 = (plsc.load_gather(ev_v, [kiota * 16 + eb])
                      + plsc.load_gather(ev_v, [kiota * 16 + eb + 1])
                      + plsc.load_gather(ev_v, [kiota * 16 + eb + 2]))
                wout_v[pl.ds(q * KNN + g2 * 16, 16)] = 1.0 / (1.0 + d2 * inv_r2)

        def chunk(ci, carry):
            q0 = base + ci * QB
            pltpu.sync_copy(qp_h.at[pl.ds(q0 * 16, QB * 16)], qp_v)
            dmas = []
            for g in range(2):
                pltpu.sync_copy(nb_h.at[pl.ds(q0 * KNN + g * 128, 128)],
                                idx_bufs[g])
                dmas.append(pltpu.async_copy(sp_h.at[idx_bufs[g]],
                                             row_bufs[g], sems[g]))
            for g in range(2):
                dmas[g].wait()
                for q in range(g * 4, g * 4 + 4):
                    compute(q, row_bufs[g], q0)
            pltpu.sync_copy(wout_v, w_h.at[pl.ds(q0 * KNN, G)])
            return carry

        lax.fori_loop(0, nchunk, chunk, 0)

    return body(qflat, spad, nb_flat)


_WPOOL_QB = {16: 4, 32: 4, 64: 4, 128: 4, 256: 2}


def _sc_wpool(feats, nb_flat, w_flat, C):
    CP = feats.shape[1]
    Mp = w_flat.shape[0] // KNN
    QB = _WPOOL_QB[C]
    G = QB * KNN
    rows_pw = Mp // NW
    nchunk = rows_pw // QB
    scale = 1.0 / KNN

    @functools.partial(
        pl.kernel,
        out_type=jax.ShapeDtypeStruct((Mp * C,), jnp.float32),
        mesh=_sc_mesh(),
        compiler_params=_CP,
        scratch_types=[
            pltpu.VMEM((G,), jnp.int32),
            pltpu.VMEM((G,), jnp.int32),
            pltpu.VMEM((G,), jnp.float32),
            pltpu.VMEM((G,), jnp.float32),
            pltpu.VMEM((G, CP), jnp.float32),
            pltpu.VMEM((G, CP), jnp.float32),
            pltpu.VMEM((QB * C,), jnp.float32),
            pltpu.VMEM((QB * C,), jnp.float32),
            pltpu.SemaphoreType.DMA,
            pltpu.SemaphoreType.DMA,
        ])
    def body(f_h, nb_h, w_h, out_h, idxA, idxB, wA, wB, rowsA, rowsB,
             outA, outB, semA, semB):
        base = _wid() * rows_pw
        kiota = lax.broadcasted_iota(jnp.int32, (16,), 0)
        bufs = [(idxA, wA, rowsA, outA, semA), (idxB, wB, rowsB, outB, semB)]

        def compute(w_v, rows_v, out_v):
            for q in range(QB):
                accs = [jnp.zeros((16,), jnp.float32) for _ in range(C // 16)]
                for k in range(KNN):
                    wk = plsc.load_gather(w_v, [_full(q * KNN + k)])
                    row = q * KNN + k
                    for cc in range(C // 16):
                        rv = plsc.load_gather(rows_v, [_full(row), kiota + cc * 16])
                        accs[cc] = accs[cc] + wk * rv
                for cc in range(C // 16):
                    out_v[pl.ds(q * C + cc * 16, 16)] = accs[cc] * scale

        def pair(pi, carry):
            dmas = []
            for g in range(2):
                q0 = base + (2 * pi + g) * QB
                idx_v, w_v, rows_v, out_v, sem = bufs[g]
                pltpu.sync_copy(nb_h.at[pl.ds(q0 * KNN, G)], idx_v)
                pltpu.sync_copy(w_h.at[pl.ds(q0 * KNN, G)], w_v)
                dmas.append(pltpu.async_copy(f_h.at[idx_v], rows_v, sem))
            for g in range(2):
                q0 = base + (2 * pi + g) * QB
                idx_v, w_v, rows_v, out_v, sem = bufs[g]
                dmas[g].wait()
                compute(w_v, rows_v, out_v)
                pltpu.sync_copy(out_v, out_h.at[pl.ds(q0 * C, QB * C)])
            return carry

        lax.fori_loop(0, nchunk // 2, pair, 0)

    return body(feats, nb_flat, w_flat).reshape(Mp, C)


def _wpool_any(feats, nb_flat, w_flat):
    Ns, C = feats.shape
    if C < 128:
        feats = jnp.pad(feats, ((0, 0), (0, 128 - C)))
    return _sc_wpool(feats, nb_flat, w_flat, C)


_MAXP_QB = {128: 4, 256: 2, 512: 1}


def _sc_maxpool(feats, nb_flat, Mp):
    C = feats.shape[1]
    QB = _MAXP_QB[C]
    G = QB * KNN
    rows_pw = Mp // NW
    nchunk = rows_pw // QB

    @functools.partial(
        pl.kernel,
        out_type=jax.ShapeDtypeStruct((Mp * C,), jnp.float32),
        mesh=_sc_mesh(),
        scratch_types=[
            pltpu.VMEM((G,), jnp.int32),
            pltpu.VMEM((G, C), jnp.float32),
            pltpu.VMEM((QB * C,), jnp.float32),
            pltpu.SemaphoreType.DMA,
        ])
    def body(f_h, nb_h, out_h, idx_v, rows_v, out_v, sem):
        base = _wid() * rows_pw

        def chunk(ci, carry):
            q0 = base + ci * QB
            pltpu.sync_copy(nb_h.at[pl.ds(q0 * KNN, G)], idx_v)
            pltpu.async_copy(f_h.at[idx_v], rows_v, sem).wait()
            for q in range(QB):
                for cc in range(C // 16):
                    acc = rows_v[q * KNN, pl.ds(cc * 16, 16)]
                    for k in range(1, KNN):
                        acc = jnp.maximum(acc, rows_v[q * KNN + k, pl.ds(cc * 16, 16)])
                    out_v[pl.ds(q * C + cc * 16, 16)] = acc
            pltpu.sync_copy(out_v, out_h.at[pl.ds(q0 * C, QB * C)])
            return carry

        lax.fori_loop(0, nchunk, chunk, 0)

    return body(feats, nb_flat).reshape(Mp, C)


def _sc_rowgather(table, idx_flat):
    C = table.shape[1]
    Mp = idx_flat.shape[0]
    QB = 8
    rows_pw = Mp // NW
    nchunk = rows_pw // QB

    @functools.partial(
        pl.kernel,
        out_type=jax.ShapeDtypeStruct((Mp, C), jnp.float32),
        mesh=_sc_mesh(),
        scratch_types=[
            pltpu.VMEM((QB,), jnp.int32),
            pltpu.VMEM((QB, C), jnp.float32),
            pltpu.SemaphoreType.DMA,
        ])
    def body(t_h, i_h, out_h, idx_v, rows_v, sem):
        base = _wid() * rows_pw

        def chunk(ci, carry):
            q0 = base + ci * QB
            pltpu.sync_copy(i_h.at[pl.ds(q0, QB)], idx_v)
            pltpu.async_copy(t_h.at[idx_v], rows_v, sem).wait()
            pltpu.sync_copy(rows_v, out_h.at[pl.ds(q0, QB)])
            return carry

        lax.fori_loop(0, nchunk, chunk, 0)

    return body(table, idx_flat)


def _lanepad16(x):
    return jnp.pad(x, ((0, 0), (0, 16 - x.shape[1])))


def kernel(feats, points0, points1, points2, points3,
           neighbors0, neighbors1, neighbors2, neighbors3,
           subsampling0, subsampling1, subsampling2,
           upsampling1, upsampling2, params):
    p = params
    N1, N2, N3, N4 = points0.shape[0], points1.shape[0], points2.shape[0], points3.shape[0]
    Mp1, Mp2, Mp3, Mp4 = _padM(N1), _padM(N2), _padM(N3), _padM(N4)

    nb0 = _pad_rows(neighbors0, ROW_ALIGN).reshape(-1)
    nb1 = _pad_rows(neighbors1, ROW_ALIGN).reshape(-1)
    nb2 = _pad_rows(neighbors2, ROW_ALIGN).reshape(-1)
    nb3 = _pad_rows(neighbors3, ROW_ALIGN).reshape(-1)
    ss0 = _pad_rows(subsampling0, ROW_ALIGN).reshape(-1)
    ss1 = _pad_rows(subsampling1, ROW_ALIGN).reshape(-1)
    ss2 = _pad_rows(subsampling2, ROW_ALIGN).reshape(-1)

    sp0, sp1, sp2, sp3 = (jnp.pad(x, ((0, 0), (0, 125)))
                          for x in (points0, points1, points2, points3))
    qp0, qp1, qp2, qp3 = (_pad_rows(_lanepad16(x), ROW_ALIGN)
                          for x in (points0, points1, points2, points3))

    w_n0 = _sc_weights(qp0, sp0, nb0)
    w_s0 = _sc_weights(qp1, sp0, ss0)
    w_n1 = _sc_weights(qp1, sp1, nb1)
    w_s1 = _sc_weights(qp2, sp1, ss1)
    w_n2 = _sc_weights(qp2, sp2, nb2)
    w_s2 = _sc_weights(qp3, sp2, ss2)
    w_n3 = _sc_weights(qp3, sp3, nb3)

    wp = _wpool_any(_lanepad16(feats), nb0, w_n0)[:N1, :8]
    W8 = jnp.pad(p["e11"]["W"], ((0, 7), (0, 0)))
    f1 = _mm(wp, W8, act=True)

    def resnet(f, Mq, wtab, nbf, Mp, pr, strided):
        x = _mm(f, pr["Wa"], act=True)
        xp = _wpool_any(x, nbf, wtab)[:Mq]
        short = _sc_maxpool(f, nbf, Mp)[:Mq] if strided else f
        return _tail(xp, short, pr["Wb"], pr["Wc"], pr.get("Ws"))

    f1 = resnet(f1, N1, w_n0, nb0, Mp1, p["e12"], False)
    f2 = resnet(f1, N2, w_s0, ss0, Mp2, p["e21"], True)
    f2 = resnet(f2, N2, w_n1, nb1, Mp2, p["e22"], False)
    f2 = resnet(f2, N2, w_n1, nb1, Mp2, p["e23"], False)
    f3 = resnet(f2, N3, w_s1, ss1, Mp3, p["e31"], True)
    f3 = resnet(f3, N3, w_n2, nb2, Mp3, p["e32"], False)
    f3 = resnet(f3, N3, w_n2, nb2, Mp3, p["e33"], False)
    f4 = resnet(f3, N4, w_s2, ss2, Mp4, p["e41"], True)
    f4 = resnet(f4, N4, w_n3, nb3, Mp4, p["e42"], False)
    f4 = resnet(f4, N4, w_n3, nb3, Mp4, p["e43"], False)

    up2 = _pad_rows(upsampling2[:, 0], ROW_ALIGN)
    up1 = _pad_rows(upsampling1[:, 0], ROW_ALIGN)
    g3 = _sc_rowgather(f4, up2)[:N3]
    l3 = _dec(g3, f3, p["d3"]["W"], act=True)
    g2 = _sc_rowgather(l3, up1)[:N2]
    l2 = _dec(g2, f2, p["d2"]["W"], act=False)
    return (l2, l3, f4)


# speedup vs baseline: 2.8828x; 1.0965x over previous
import functools
import jax
import jax.numpy as jnp
from jax import lax
from jax.experimental import pallas as pl
from jax.experimental.pallas import tpu as pltpu
from jax.experimental.pallas import tpu_sc as plsc

RADIUS = 0.0625
KNN = 32
BR = 512
NC, NS = 2, 16
NW = NC * NS
ROW_ALIGN = 256

_CP = pltpu.CompilerParams(needs_layout_passes=False)


def _lrelu(x):
    return jnp.where(x >= 0, x, 0.1 * x)


def _pad_rows(x, mult):
    p = (-x.shape[0]) % mult
    if p:
        x = jnp.pad(x, ((0, p),) + ((0, 0),) * (x.ndim - 1))
    return x


def _padM(M):
    return -(-M // ROW_ALIGN) * ROW_ALIGN


def _mm_kernel(x_ref, w_ref, o_ref, *, act):
    y = jnp.dot(x_ref[...], w_ref[...], preferred_element_type=jnp.float32)
    o_ref[...] = _lrelu(y) if act else y


def _mm(x, W, act=True):
    M, Ci = x.shape
    Co = W.shape[1]
    xp = _pad_rows(x, BR)
    Mp = xp.shape[0]
    out = pl.pallas_call(
        functools.partial(_mm_kernel, act=act),
        grid=(Mp // BR,),
        in_specs=[pl.BlockSpec((BR, Ci), lambda i: (i, 0)),
                  pl.BlockSpec((Ci, Co), lambda i: (0, 0))],
        out_specs=pl.BlockSpec((BR, Co), lambda i: (i, 0)),
        out_shape=jax.ShapeDtypeStruct((Mp, Co), jnp.float32),
    )(xp, W)
    return out[:M]


def _tail_kernel(x_ref, s_ref, wb_ref, wc_ref, ws_ref, o_ref):
    h = _lrelu(jnp.dot(x_ref[...], wb_ref[...], preferred_element_type=jnp.float32))
    y = jnp.dot(h, wc_ref[...], preferred_element_type=jnp.float32)
    sh = s_ref[...]
    if ws_ref is not None:
        sh = jnp.dot(sh, ws_ref[...], preferred_element_type=jnp.float32)
    o_ref[...] = _lrelu(y + sh)


def _tail_kernel_nows(x_ref, s_ref, wb_ref, wc_ref, o_ref):
    _tail_kernel(x_ref, s_ref, wb_ref, wc_ref, None, o_ref)


def _tail(x, short, Wb, Wc, Ws=None):
    M, m = x.shape
    o = Wc.shape[1]
    si = short.shape[1]
    xp = _pad_rows(x, BR)
    sp = _pad_rows(short, BR)
    Mp = xp.shape[0]
    specs = [pl.BlockSpec((BR, m), lambda i: (i, 0)),
             pl.BlockSpec((BR, si), lambda i: (i, 0)),
             pl.BlockSpec((m, m), lambda i: (0, 0)),
             pl.BlockSpec((m, o), lambda i: (0, 0))]
    args = [xp, sp, Wb, Wc]
    if Ws is not None:
        specs.append(pl.BlockSpec((si, o), lambda i: (0, 0)))
        args.append(Ws)
        body = _tail_kernel
    else:
        body = _tail_kernel_nows
    out = pl.pallas_call(
        body,
        grid=(Mp // BR,),
        in_specs=specs,
        out_specs=pl.BlockSpec((BR, o), lambda i: (i, 0)),
        out_shape=jax.ShapeDtypeStruct((Mp, o), jnp.float32),
    )(*args)
    return out[:M]


def _dec_kernel(g_ref, f_ref, w1_ref, w2_ref, o_ref, *, act):
    y = jnp.dot(g_ref[...], w1_ref[...], preferred_element_type=jnp.float32)
    y += jnp.dot(f_ref[...], w2_ref[...], preferred_element_type=jnp.float32)
    o_ref[...] = _lrelu(y) if act else y


def _dec(g, f, W, act):
    M, cg = g.shape
    cf = f.shape[1]
    co = W.shape[1]
    W1, W2 = W[:cg], W[cg:]
    gp = _pad_rows(g, BR)
    fp = _pad_rows(f, BR)
    Mp = gp.shape[0]
    out = pl.pallas_call(
        functools.partial(_dec_kernel, act=act),
        grid=(Mp // BR,),
        in_specs=[pl.BlockSpec((BR, cg), lambda i: (i, 0)),
                  pl.BlockSpec((BR, cf), lambda i: (i, 0)),
                  pl.BlockSpec((cg, co), lambda i: (0, 0)),
                  pl.BlockSpec((cf, co), lambda i: (0, 0))],
        out_specs=pl.BlockSpec((BR, co), lambda i: (i, 0)),
        out_shape=jax.ShapeDtypeStruct((Mp, co), jnp.float32),
    )(gp, fp, W1, W2)
    return out[:M]


def _sc_mesh():
    return plsc.VectorSubcoreMesh(core_axis_name="c", subcore_axis_name="s")


def _wid():
    return lax.axis_index("s") * NC + lax.axis_index("c")


def _full(v):
    return jnp.full((16,), v, jnp.int32)


def _sc_weights(qpad, spad, nb_flat):
    Mp = qpad.shape[0]
    qflat = qpad.reshape(-1)
    QB = 8
    G = QB * KNN
    rows_pw = Mp // NW
    nchunk = rows_pw // QB
    inv_r2 = 1.0 / (RADIUS * RADIUS)

    @functools.partial(
        pl.kernel,
        out_type=jax.ShapeDtypeStruct((Mp * KNN,), jnp.float32),
        mesh=_sc_mesh(),
        compiler_params=_CP,
        scratch_types=[
            pltpu.VMEM((128,), jnp.int32),
            pltpu.VMEM((128,), jnp.int32),
            pltpu.VMEM((QB * 16,), jnp.float32),
            pltpu.VMEM((128, 128), jnp.float32),
            pltpu.VMEM((128, 128), jnp.float32),
            pltpu.VMEM((KNN * 16,), jnp.float32),
            pltpu.VMEM((G,), jnp.float32),
            pltpu.SemaphoreType.DMA,
            pltpu.SemaphoreType.DMA,
        ])
    def body(qp_h, sp_h, nb_h, w_h, idx0_v, idx1_v, qp_v, rows0_v,
             rows1_v, ev_v, wout_v, sem0, sem1):
        base = _wid() * rows_pw
        kiota = lax.broadcasted_iota(jnp.int32, (16,), 0)
        zero16 = jnp.zeros((16,), jnp.float32)
        idx_bufs = [idx0_v, idx1_v]
        row_bufs = [rows0_v, rows1_v]
        sems = [sem0, sem1]

        def compute(q, buf, q0):
            qx = plsc.load_gather(qp_v, [_full(q * 16 + 0)])
            qy = plsc.load_gather(qp_v, [_full(q * 16 + 1)])
            qz = plsc.load_gather(qp_v, [_full(q * 16 + 2)])
            qvec = jnp.where(kiota == 0, qx,
                             jnp.where(kiota == 1, qy,
                                       jnp.where(kiota == 2, qz, zero16)))
            for j in range(KNN):
                v = plsc.load_gather(buf, [_full((q % 4) * KNN + j), kiota])
                d = v - qvec
                ev_v[pl.ds(j * 16, 16)] = d * d
            for g2 in range(KNN // 16):
                eb = g2 * 256
                d2 = (plsc.load_gather(ev_v, [kiota * 16 + eb])
                      + plsc.load_gather(ev_v, [kiota * 16 + eb + 1])
                      + plsc.load_gather(ev_v, [kiota * 16 + eb + 2]))
                wout_v[pl.ds(q * KNN + g2 * 16, 16)] = 1.0 / (1.0 + d2 * inv_r2)

        def chunk(ci, carry):
            q0 = base + ci * QB
            pltpu.sync_copy(qp_h.at[pl.ds(q0 * 16, QB * 16)], qp_v)
            dmas = []
            for g in range(2):
                pltpu.sync_copy(nb_h.at[pl.ds(q0 * KNN + g * 128, 128)],
                                idx_bufs[g])
                dmas.append(pltpu.async_copy(sp_h.at[idx_bufs[g]],
                                             row_bufs[g], sems[g]))
            for g in range(2):
                dmas[g].wait()
                for q in range(g * 4, g * 4 + 4):
                    compute(q, row_bufs[g], q0)
            pltpu.sync_copy(wout_v, w_h.at[pl.ds(q0 * KNN, G)])
            return carry

        lax.fori_loop(0, nchunk, chunk, 0)

    return body(qflat, spad, nb_flat)


_WPOOL_QB = {16: 4, 32: 4, 64: 4, 128: 4, 256: 2}


def _sc_wpool(feats, nb_flat, w_flat, C):
    CP = feats.shape[1]
    Mp = w_flat.shape[0] // KNN
    QB = _WPOOL_QB[C]
    G = QB * KNN
    rows_pw = Mp // NW
    nchunk = rows_pw // QB
    scale = 1.0 / KNN

    @functools.partial(
        pl.kernel,
        out_type=jax.ShapeDtypeStruct((Mp * C,), jnp.float32),
        mesh=_sc_mesh(),
        compiler_params=_CP,
        scratch_types=[
            pltpu.VMEM((G,), jnp.int32),
            pltpu.VMEM((G,), jnp.int32),
            pltpu.VMEM((G,), jnp.float32),
            pltpu.VMEM((G,), jnp.float32),
            pltpu.VMEM((G, CP), jnp.float32),
            pltpu.VMEM((G, CP), jnp.float32),
            pltpu.VMEM((QB * C,), jnp.float32),
            pltpu.VMEM((QB * C,), jnp.float32),
            pltpu.SemaphoreType.DMA,
            pltpu.SemaphoreType.DMA,
        ])
    def body(f_h, nb_h, w_h, out_h, idxA, idxB, wA, wB, rowsA, rowsB,
             outA, outB, semA, semB):
        base = _wid() * rows_pw
        kiota = lax.broadcasted_iota(jnp.int32, (16,), 0)
        bufs = [(idxA, wA, rowsA, outA, semA), (idxB, wB, rowsB, outB, semB)]

        def compute(w_v, rows_v, out_v):
            for q in range(QB):
                accs = [jnp.zeros((16,), jnp.float32) for _ in range(C // 16)]
                for k in range(KNN):
                    wk = plsc.load_gather(w_v, [_full(q * KNN + k)])
                    row = q * KNN + k
                    for cc in range(C // 16):
                        rv = plsc.load_gather(rows_v, [_full(row), kiota + cc * 16])
                        accs[cc] = accs[cc] + wk * rv
                for cc in range(C // 16):
                    out_v[pl.ds(q * C + cc * 16, 16)] = accs[cc] * scale

        def pair(pi, carry):
            dmas = []
            for g in range(2):
                q0 = base + (2 * pi + g) * QB
                idx_v, w_v, rows_v, out_v, sem = bufs[g]
                pltpu.sync_copy(nb_h.at[pl.ds(q0 * KNN, G)], idx_v)
                pltpu.sync_copy(w_h.at[pl.ds(q0 * KNN, G)], w_v)
                dmas.append(pltpu.async_copy(f_h.at[idx_v], rows_v, sem))
            for g in range(2):
                q0 = base + (2 * pi + g) * QB
                idx_v, w_v, rows_v, out_v, sem = bufs[g]
                dmas[g].wait()
                compute(w_v, rows_v, out_v)
                pltpu.sync_copy(out_v, out_h.at[pl.ds(q0 * C, QB * C)])
            return carry

        lax.fori_loop(0, nchunk // 2, pair, 0)

    return body(feats, nb_flat, w_flat).reshape(Mp, C)


def _sc_wpool_c1(fflat, nb_flat, w_flat):
    Nf = fflat.shape[0]
    Mp = w_flat.shape[0] // KNN
    QB = 16
    G = QB * KNN
    rows_pw = Mp // NW
    nchunk = rows_pw // QB
    scale = 1.0 / KNN

    @functools.partial(
        pl.kernel,
        out_type=jax.ShapeDtypeStruct((Mp,), jnp.float32),
        mesh=_sc_mesh(),
        compiler_params=_CP,
        scratch_types=[
            pltpu.VMEM((Nf,), jnp.float32),
            pltpu.VMEM((G,), jnp.int32),
            pltpu.VMEM((G,), jnp.float32),
            pltpu.VMEM((QB,), jnp.float32),
            pltpu.SemaphoreType.DMA,
        ])
    def body(f_h, nb_h, w_h, out_h, f_v, idx_v, w_v, out_v, sem):
        pltpu.sync_copy(f_h, f_v)
        base = _wid() * rows_pw
        kiota = lax.broadcasted_iota(jnp.int32, (16,), 0)

        def chunk(ci, carry):
            q0 = base + ci * QB
            pltpu.sync_copy(nb_h.at[pl.ds(q0 * KNN, G)], idx_v)
            pltpu.sync_copy(w_h.at[pl.ds(q0 * KNN, G)], w_v)
            acc = jnp.zeros((16,), jnp.float32)
            for j in range(KNN):
                iv = plsc.load_gather(idx_v, [kiota * KNN + j])
                fv = plsc.load_gather(f_v, [iv])
                wv = plsc.load_gather(w_v, [kiota * KNN + j])
                acc = acc + fv * wv
            out_v[...] = acc * scale
            pltpu.sync_copy(out_v, out_h.at[pl.ds(q0, QB)])
            return carry

        lax.fori_loop(0, nchunk, chunk, 0)

    return body(fflat, nb_flat, w_flat)


def _wpool_any(feats, nb_flat, w_flat):
    Ns, C = feats.shape
    if C < 128:
        feats = jnp.pad(feats, ((0, 0), (0, 128 - C)))
    return _sc_wpool(feats, nb_flat, w_flat, C)


_MAXP_QB = {128: 4, 256: 2, 512: 1}


def _sc_maxpool(feats, nb_flat, Mp):
    C = feats.shape[1]
    QB = _MAXP_QB[C]
    G = QB * KNN
    rows_pw = Mp // NW
    nchunk = rows_pw // QB

    @functools.partial(
        pl.kernel,
        out_type=jax.ShapeDtypeStruct((Mp * C,), jnp.float32),
        mesh=_sc_mesh(),
        scratch_types=[
            pltpu.VMEM((G,), jnp.int32),
            pltpu.VMEM((G, C), jnp.float32),
            pltpu.VMEM((QB * C,), jnp.float32),
            pltpu.SemaphoreType.DMA,
        ])
    def body(f_h, nb_h, out_h, idx_v, rows_v, out_v, sem):
        base = _wid() * rows_pw

        def chunk(ci, carry):
            q0 = base + ci * QB
            pltpu.sync_copy(nb_h.at[pl.ds(q0 * KNN, G)], idx_v)
            pltpu.async_copy(f_h.at[idx_v], rows_v, sem).wait()
            for q in range(QB):
                for cc in range(C // 16):
                    acc = rows_v[q * KNN, pl.ds(cc * 16, 16)]
                    for k in range(1, KNN):
                        acc = jnp.maximum(acc, rows_v[q * KNN + k, pl.ds(cc * 16, 16)])
                    out_v[pl.ds(q * C + cc * 16, 16)] = acc
            pltpu.sync_copy(out_v, out_h.at[pl.ds(q0 * C, QB * C)])
            return carry

        lax.fori_loop(0, nchunk, chunk, 0)

    return body(feats, nb_flat).reshape(Mp, C)


def _sc_rowgather(table, idx_flat):
    C = table.shape[1]
    Mp = idx_flat.shape[0]
    QB = 8
    rows_pw = Mp // NW
    nchunk = rows_pw // QB

    @functools.partial(
        pl.kernel,
        out_type=jax.ShapeDtypeStruct((Mp, C), jnp.float32),
        mesh=_sc_mesh(),
        scratch_types=[
            pltpu.VMEM((QB,), jnp.int32),
            pltpu.VMEM((QB, C), jnp.float32),
            pltpu.SemaphoreType.DMA,
        ])
    def body(t_h, i_h, out_h, idx_v, rows_v, sem):
        base = _wid() * rows_pw

        def chunk(ci, carry):
            q0 = base + ci * QB
            pltpu.sync_copy(i_h.at[pl.ds(q0, QB)], idx_v)
            pltpu.async_copy(t_h.at[idx_v], rows_v, sem).wait()
            pltpu.sync_copy(rows_v, out_h.at[pl.ds(q0, QB)])
            return carry

        lax.fori_loop(0, nchunk, chunk, 0)

    return body(table, idx_flat)


def _lanepad16(x):
    return jnp.pad(x, ((0, 0), (0, 16 - x.shape[1])))


def kernel(feats, points0, points1, points2, points3,
           neighbors0, neighbors1, neighbors2, neighbors3,
           subsampling0, subsampling1, subsampling2,
           upsampling1, upsampling2, params):
    p = params
    N1, N2, N3, N4 = points0.shape[0], points1.shape[0], points2.shape[0], points3.shape[0]
    Mp1, Mp2, Mp3, Mp4 = _padM(N1), _padM(N2), _padM(N3), _padM(N4)

    nb0 = _pad_rows(neighbors0, ROW_ALIGN).reshape(-1)
    nb1 = _pad_rows(neighbors1, ROW_ALIGN).reshape(-1)
    nb2 = _pad_rows(neighbors2, ROW_ALIGN).reshape(-1)
    nb3 = _pad_rows(neighbors3, ROW_ALIGN).reshape(-1)
    ss0 = _pad_rows(subsampling0, ROW_ALIGN).reshape(-1)
    ss1 = _pad_rows(subsampling1, ROW_ALIGN).reshape(-1)
    ss2 = _pad_rows(subsampling2, ROW_ALIGN).reshape(-1)

    sp0, sp1, sp2, sp3 = (jnp.pad(x, ((0, 0), (0, 125)))
                          for x in (points0, points1, points2, points3))
    qp0, qp1, qp2, qp3 = (_pad_rows(_lanepad16(x), ROW_ALIGN)
                          for x in (points0, points1, points2, points3))

    w_n0 = _sc_weights(qp0, sp0, nb0)
    w_s0 = _sc_weights(qp1, sp0, ss0)
    w_n1 = _sc_weights(qp1, sp1, nb1)
    w_s1 = _sc_weights(qp2, sp1, ss1)
    w_n2 = _sc_weights(qp2, sp2, nb2)
    w_s2 = _sc_weights(qp3, sp2, ss2)
    w_n3 = _sc_weights(qp3, sp3, nb3)

    f0 = jnp.pad(feats[:, 0], (0, (-N1) % 8))
    wp1 = _sc_wpool_c1(f0, nb0, w_n0)[:N1]
    wp = jnp.pad(wp1[:, None], ((0, 0), (0, 7)))
    W8 = jnp.pad(p["e11"]["W"], ((0, 7), (0, 0)))
    f1 = _mm(wp, W8, act=True)

    def resnet(f, Mq, wtab, nbf, Mp, pr, strided):
        x = _mm(f, pr["Wa"], act=True)
        xp = _wpool_any(x, nbf, wtab)[:Mq]
        short = _sc_maxpool(f, nbf, Mp)[:Mq] if strided else f
        return _tail(xp, short, pr["Wb"], pr["Wc"], pr.get("Ws"))

    f1 = resnet(f1, N1, w_n0, nb0, Mp1, p["e12"], False)
    f2 = resnet(f1, N2, w_s0, ss0, Mp2, p["e21"], True)
    f2 = resnet(f2, N2, w_n1, nb1, Mp2, p["e22"], False)
    f2 = resnet(f2, N2, w_n1, nb1, Mp2, p["e23"], False)
    f3 = resnet(f2, N3, w_s1, ss1, Mp3, p["e31"], True)
    f3 = resnet(f3, N3, w_n2, nb2, Mp3, p["e32"], False)
    f3 = resnet(f3, N3, w_n2, nb2, Mp3, p["e33"], False)
    f4 = resnet(f3, N4, w_s2, ss2, Mp4, p["e41"], True)
    f4 = resnet(f4, N4, w_n3, nb3, Mp4, p["e42"], False)
    f4 = resnet(f4, N4, w_n3, nb3, Mp4, p["e43"], False)

    up2 = _pad_rows(upsampling2[:, 0], ROW_ALIGN)
    up1 = _pad_rows(upsampling1[:, 0], ROW_ALIGN)
    g3 = _sc_rowgather(f4, up2)[:N3]
    l3 = _dec(g3, f3, p["d3"]["W"], act=True)
    g2 = _sc_rowgather(l3, up1)[:N2]
    l2 = _dec(g2, f2, p["d2"]["W"], act=False)
    return (l2, l3, f4)


# speedup vs baseline: 3.7533x; 1.3020x over previous
import functools
import jax
import jax.numpy as jnp
from jax import lax
from jax.experimental import pallas as pl
from jax.experimental.pallas import tpu as pltpu
from jax.experimental.pallas import tpu_sc as plsc

RADIUS = 0.0625
KNN = 32
BR = 512
NC, NS = 2, 16
NW = NC * NS
ROW_ALIGN = 256

_CP = pltpu.CompilerParams(needs_layout_passes=False)


def _lrelu(x):
    return jnp.where(x >= 0, x, 0.1 * x)


def _pad_rows(x, mult):
    p = (-x.shape[0]) % mult
    if p:
        x = jnp.pad(x, ((0, p),) + ((0, 0),) * (x.ndim - 1))
    return x


def _padM(M):
    return -(-M // ROW_ALIGN) * ROW_ALIGN


def _mm_kernel(x_ref, w_ref, o_ref, *, act):
    y = jnp.dot(x_ref[...], w_ref[...], preferred_element_type=jnp.float32)
    o_ref[...] = _lrelu(y) if act else y


def _mm(x, W, act=True):
    M, Ci = x.shape
    Co = W.shape[1]
    xp = _pad_rows(x, BR)
    Mp = xp.shape[0]
    out = pl.pallas_call(
        functools.partial(_mm_kernel, act=act),
        grid=(Mp // BR,),
        in_specs=[pl.BlockSpec((BR, Ci), lambda i: (i, 0)),
                  pl.BlockSpec((Ci, Co), lambda i: (0, 0))],
        out_specs=pl.BlockSpec((BR, Co), lambda i: (i, 0)),
        out_shape=jax.ShapeDtypeStruct((Mp, Co), jnp.float32),
    )(xp, W)
    return out[:M]


def _tail_kernel(x_ref, s_ref, wb_ref, wc_ref, ws_ref, o_ref):
    h = _lrelu(jnp.dot(x_ref[...], wb_ref[...], preferred_element_type=jnp.float32))
    y = jnp.dot(h, wc_ref[...], preferred_element_type=jnp.float32)
    sh = s_ref[...]
    if ws_ref is not None:
        sh = jnp.dot(sh, ws_ref[...], preferred_element_type=jnp.float32)
    o_ref[...] = _lrelu(y + sh)


def _tail_kernel_nows(x_ref, s_ref, wb_ref, wc_ref, o_ref):
    _tail_kernel(x_ref, s_ref, wb_ref, wc_ref, None, o_ref)


def _tail(x, short, Wb, Wc, Ws=None):
    M, m = x.shape
    o = Wc.shape[1]
    si = short.shape[1]
    xp = _pad_rows(x, BR)
    sp = _pad_rows(short, BR)
    Mp = xp.shape[0]
    specs = [pl.BlockSpec((BR, m), lambda i: (i, 0)),
             pl.BlockSpec((BR, si), lambda i: (i, 0)),
             pl.BlockSpec((m, m), lambda i: (0, 0)),
             pl.BlockSpec((m, o), lambda i: (0, 0))]
    args = [xp, sp, Wb, Wc]
    if Ws is not None:
        specs.append(pl.BlockSpec((si, o), lambda i: (0, 0)))
        args.append(Ws)
        body = _tail_kernel
    else:
        body = _tail_kernel_nows
    out = pl.pallas_call(
        body,
        grid=(Mp // BR,),
        in_specs=specs,
        out_specs=pl.BlockSpec((BR, o), lambda i: (i, 0)),
        out_shape=jax.ShapeDtypeStruct((Mp, o), jnp.float32),
    )(*args)
    return out[:M]


def _dec_kernel(g_ref, f_ref, w1_ref, w2_ref, o_ref, *, act):
    y = jnp.dot(g_ref[...], w1_ref[...], preferred_element_type=jnp.float32)
    y += jnp.dot(f_ref[...], w2_ref[...], preferred_element_type=jnp.float32)
    o_ref[...] = _lrelu(y) if act else y


def _dec(g, f, W, act):
    M, cg = g.shape
    cf = f.shape[1]
    co = W.shape[1]
    W1, W2 = W[:cg], W[cg:]
    gp = _pad_rows(g, BR)
    fp = _pad_rows(f, BR)
    Mp = gp.shape[0]
    out = pl.pallas_call(
        functools.partial(_dec_kernel, act=act),
        grid=(Mp // BR,),
        in_specs=[pl.BlockSpec((BR, cg), lambda i: (i, 0)),
                  pl.BlockSpec((BR, cf), lambda i: (i, 0)),
                  pl.BlockSpec((cg, co), lambda i: (0, 0)),
                  pl.BlockSpec((cf, co), lambda i: (0, 0))],
        out_specs=pl.BlockSpec((BR, co), lambda i: (i, 0)),
        out_shape=jax.ShapeDtypeStruct((Mp, co), jnp.float32),
    )(gp, fp, W1, W2)
    return out[:M]


def _sc_mesh():
    return plsc.VectorSubcoreMesh(core_axis_name="c", subcore_axis_name="s")


def _wid():
    return lax.axis_index("s") * NC + lax.axis_index("c")


def _full(v):
    return jnp.full((16,), v, jnp.int32)


def _sc_weights_pl(qpad, planes, nb_flat):
    Mp = qpad.shape[0]
    qflat = qpad.reshape(-1)
    Ns8 = planes.shape[1]
    QB = 16
    G = QB * KNN
    rows_pw = Mp // NW
    nchunk = rows_pw // QB
    inv_r2 = 1.0 / (RADIUS * RADIUS)

    def make_pass(last):
        nplane = 1 if last else 2

        @functools.partial(
            pl.kernel,
            out_type=jax.ShapeDtypeStruct((Mp * KNN,), jnp.float32),
            mesh=_sc_mesh(),
            compiler_params=_CP,
            scratch_types=[pltpu.VMEM((Ns8,), jnp.float32)] * nplane + [
                pltpu.VMEM((QB * 16,), jnp.float32),
                pltpu.VMEM((G,), jnp.int32),
                pltpu.VMEM((G,), jnp.float32),
                pltpu.VMEM((G,), jnp.float32),
                pltpu.SemaphoreType.DMA,
            ])
        def body(*refs):
            if last:
                (pz_h, qp_h, nb_h, part_h, w_h,
                 pz_v, qp_v, idx_v, part_v, tile_v, sem) = refs
                planes_v = [pz_v]
                pltpu.sync_copy(pz_h, pz_v)
            else:
                (px_h, py_h, qp_h, nb_h, w_h,
                 px_v, py_v, qp_v, idx_v, part_v, tile_v, sem) = refs
                planes_v = [px_v, py_v]
                pltpu.sync_copy(px_h, px_v)
                pltpu.sync_copy(py_h, py_v)
            base = _wid() * rows_pw
            kiota = lax.broadcasted_iota(jnp.int32, (16,), 0)

            def chunk(ci, carry):
                q0 = base + ci * QB
                pltpu.sync_copy(qp_h.at[pl.ds(q0 * 16, QB * 16)], qp_v)
                pltpu.sync_copy(nb_h.at[pl.ds(q0 * KNN, G)], idx_v)
                if last:
                    pltpu.sync_copy(part_h.at[pl.ds(q0 * KNN, G)], part_v)
                    qzv = plsc.load_gather(qp_v, [kiota * 16 + 2])
                    for j in range(KNN):
                        iv = plsc.load_gather(idx_v, [kiota * KNN + j])
                        dz = plsc.load_gather(planes_v[0], [iv]) - qzv
                        pj = plsc.load_gather(part_v, [kiota + j * 16])
                        d2 = pj + dz * dz
                        tile_v[pl.ds(j * 16, 16)] = 1.0 / (1.0 + d2 * inv_r2)
                    for q in range(QB):
                        for g2 in range(KNN // 16):
                            wq = plsc.load_gather(
                                tile_v, [kiota * 16 + g2 * 256 + q])
                            part_v[pl.ds(q * KNN + g2 * 16, 16)] = wq
                    pltpu.sync_copy(part_v, w_h.at[pl.ds(q0 * KNN, G)])
                else:
                    qxv = plsc.load_gather(qp_v, [kiota * 16 + 0])
                    qyv = plsc.load_gather(qp_v, [kiota * 16 + 1])
                    for j in range(KNN):
                        iv = plsc.load_gather(idx_v, [kiota * KNN + j])
                        dx = plsc.load_gather(planes_v[0], [iv]) - qxv
                        dy = plsc.load_gather(planes_v[1], [iv]) - qyv
                        tile_v[pl.ds(j * 16, 16)] = dx * dx + dy * dy
                    pltpu.sync_copy(tile_v, w_h.at[pl.ds(q0 * KNN, G)])
                return carry

            lax.fori_loop(0, nchunk, chunk, 0)

        return body

    part = make_pass(False)(planes[0], planes[1], qflat, nb_flat)
    return make_pass(True)(planes[2], qflat, nb_flat, part)


def _sc_weights(qpad, spad, nb_flat):
    Mp = qpad.shape[0]
    qflat = qpad.reshape(-1)
    QB = 8
    G = QB * KNN
    rows_pw = Mp // NW
    nchunk = rows_pw // QB
    inv_r2 = 1.0 / (RADIUS * RADIUS)

    @functools.partial(
        pl.kernel,
        out_type=jax.ShapeDtypeStruct((Mp * KNN,), jnp.float32),
        mesh=_sc_mesh(),
        compiler_params=_CP,
        scratch_types=[
            pltpu.VMEM((128,), jnp.int32),
            pltpu.VMEM((128,), jnp.int32),
            pltpu.VMEM((QB * 16,), jnp.float32),
            pltpu.VMEM((128, 128), jnp.float32),
            pltpu.VMEM((128, 128), jnp.float32),
            pltpu.VMEM((KNN * 16,), jnp.float32),
            pltpu.VMEM((G,), jnp.float32),
            pltpu.SemaphoreType.DMA,
            pltpu.SemaphoreType.DMA,
        ])
    def body(qp_h, sp_h, nb_h, w_h, idx0_v, idx1_v, qp_v, rows0_v,
             rows1_v, ev_v, wout_v, sem0, sem1):
        base = _wid() * rows_pw
        kiota = lax.broadcasted_iota(jnp.int32, (16,), 0)
        zero16 = jnp.zeros((16,), jnp.float32)
        idx_bufs = [idx0_v, idx1_v]
        row_bufs = [rows0_v, rows1_v]
        sems = [sem0, sem1]

        def compute(q, buf, q0):
            qx = plsc.load_gather(qp_v, [_full(q * 16 + 0)])
            qy = plsc.load_gather(qp_v, [_full(q * 16 + 1)])
            qz = plsc.load_gather(qp_v, [_full(q * 16 + 2)])
            qvec = jnp.where(kiota == 0, qx,
                             jnp.where(kiota == 1, qy,
                                       jnp.where(kiota == 2, qz, zero16)))
            for j in range(KNN):
                v = plsc.load_gather(buf, [_full((q % 4) * KNN + j), kiota])
                d = v - qvec
                ev_v[pl.ds(j * 16, 16)] = d * d
            for g2 in range(KNN // 16):
                eb = g2 * 256
                d2 = (plsc.load_gather(ev_v, [kiota * 16 + eb])
                      + plsc.load_gather(ev_v, [kiota * 16 + eb + 1])
                      + plsc.load_gather(ev_v, [kiota * 16 + eb + 2]))
                wout_v[pl.ds(q * KNN + g2 * 16, 16)] = 1.0 / (1.0 + d2 * inv_r2)

        def chunk(ci, carry):
            q0 = base + ci * QB
            pltpu.sync_copy(qp_h.at[pl.ds(q0 * 16, QB * 16)], qp_v)
            dmas = []
            for g in range(2):
                pltpu.sync_copy(nb_h.at[pl.ds(q0 * KNN + g * 128, 128)],
                                idx_bufs[g])
                dmas.append(pltpu.async_copy(sp_h.at[idx_bufs[g]],
                                             row_bufs[g], sems[g]))
            for g in range(2):
                dmas[g].wait()
                for q in range(g * 4, g * 4 + 4):
                    compute(q, row_bufs[g], q0)
            pltpu.sync_copy(wout_v, w_h.at[pl.ds(q0 * KNN, G)])
            return carry

        lax.fori_loop(0, nchunk, chunk, 0)

    return body(qflat, spad, nb_flat)


_WPOOL_QB = {16: 4, 32: 4, 64: 4, 128: 4, 256: 2}


def _sc_wpool(feats, nb_flat, w_flat, C):
    CP = feats.shape[1]
    Mp = w_flat.shape[0] // KNN
    QB = _WPOOL_QB[C]
    G = QB * KNN
    rows_pw = Mp // NW
    nchunk = rows_pw // QB
    scale = 1.0 / KNN

    @functools.partial(
        pl.kernel,
        out_type=jax.ShapeDtypeStruct((Mp * C,), jnp.float32),
        mesh=_sc_mesh(),
        compiler_params=_CP,
        scratch_types=[
            pltpu.VMEM((G,), jnp.int32),
            pltpu.VMEM((G,), jnp.int32),
            pltpu.VMEM((G,), jnp.float32),
            pltpu.VMEM((G,), jnp.float32),
            pltpu.VMEM((G, CP), jnp.float32),
            pltpu.VMEM((G, CP), jnp.float32),
            pltpu.VMEM((QB * C,), jnp.float32),
            pltpu.VMEM((QB * C,), jnp.float32),
            pltpu.SemaphoreType.DMA,
            pltpu.SemaphoreType.DMA,
        ])
    def body(f_h, nb_h, w_h, out_h, idxA, idxB, wA, wB, rowsA, rowsB,
             outA, outB, semA, semB):
        base = _wid() * rows_pw
        kiota = lax.broadcasted_iota(jnp.int32, (16,), 0)
        bufs = [(idxA, wA, rowsA, outA, semA), (idxB, wB, rowsB, outB, semB)]

        def compute(w_v, rows_v, out_v):
            for q in range(QB):
                accs = [jnp.zeros((16,), jnp.float32) for _ in range(C // 16)]
                for k in range(KNN):
                    wk = plsc.load_gather(w_v, [_full(q * KNN + k)])
                    row = q * KNN + k
                    for cc in range(C // 16):
                        rv = plsc.load_gather(rows_v, [_full(row), kiota + cc * 16])
                        accs[cc] = accs[cc] + wk * rv
                for cc in range(C // 16):
                    out_v[pl.ds(q * C + cc * 16, 16)] = accs[cc] * scale

        def pair(pi, carry):
            dmas = []
            for g in range(2):
                q0 = base + (2 * pi + g) * QB
                idx_v, w_v, rows_v, out_v, sem = bufs[g]
                pltpu.sync_copy(nb_h.at[pl.ds(q0 * KNN, G)], idx_v)
                pltpu.sync_copy(w_h.at[pl.ds(q0 * KNN, G)], w_v)
                dmas.append(pltpu.async_copy(f_h.at[idx_v], rows_v, sem))
            for g in range(2):
                q0 = base + (2 * pi + g) * QB
                idx_v, w_v, rows_v, out_v, sem = bufs[g]
                dmas[g].wait()
                compute(w_v, rows_v, out_v)
                pltpu.sync_copy(out_v, out_h.at[pl.ds(q0 * C, QB * C)])
            return carry

        lax.fori_loop(0, nchunk // 2, pair, 0)

    return body(feats, nb_flat, w_flat).reshape(Mp, C)


def _sc_wpool_c1(fflat, nb_flat, w_flat):
    Nf = fflat.shape[0]
    Mp = w_flat.shape[0] // KNN
    QB = 16
    G = QB * KNN
    rows_pw = Mp // NW
    nchunk = rows_pw // QB
    scale = 1.0 / KNN

    @functools.partial(
        pl.kernel,
        out_type=jax.ShapeDtypeStruct((Mp,), jnp.float32),
        mesh=_sc_mesh(),
        compiler_params=_CP,
        scratch_types=[
            pltpu.VMEM((Nf,), jnp.float32),
            pltpu.VMEM((G,), jnp.int32),
            pltpu.VMEM((G,), jnp.float32),
            pltpu.VMEM((QB,), jnp.float32),
            pltpu.SemaphoreType.DMA,
        ])
    def body(f_h, nb_h, w_h, out_h, f_v, idx_v, w_v, out_v, sem):
        pltpu.sync_copy(f_h, f_v)
        base = _wid() * rows_pw
        kiota = lax.broadcasted_iota(jnp.int32, (16,), 0)

        def chunk(ci, carry):
            q0 = base + ci * QB
            pltpu.sync_copy(nb_h.at[pl.ds(q0 * KNN, G)], idx_v)
            pltpu.sync_copy(w_h.at[pl.ds(q0 * KNN, G)], w_v)
            acc = jnp.zeros((16,), jnp.float32)
            for j in range(KNN):
                iv = plsc.load_gather(idx_v, [kiota * KNN + j])
                fv = plsc.load_gather(f_v, [iv])
                wv = plsc.load_gather(w_v, [kiota * KNN + j])
                acc = acc + fv * wv
            out_v[...] = acc * scale
            pltpu.sync_copy(out_v, out_h.at[pl.ds(q0, QB)])
            return carry

        lax.fori_loop(0, nchunk, chunk, 0)

    return body(fflat, nb_flat, w_flat)


def _wpool_any(feats, nb_flat, w_flat):
    Ns, C = feats.shape
    if C < 128:
        feats = jnp.pad(feats, ((0, 0), (0, 128 - C)))
    return _sc_wpool(feats, nb_flat, w_flat, C)


_MAXP_QB = {128: 4, 256: 2, 512: 1}


def _sc_maxpool(feats, nb_flat, Mp):
    C = feats.shape[1]
    QB = _MAXP_QB[C]
    G = QB * KNN
    rows_pw = Mp // NW
    nchunk = rows_pw // QB

    @functools.partial(
        pl.kernel,
        out_type=jax.ShapeDtypeStruct((Mp * C,), jnp.float32),
        mesh=_sc_mesh(),
        scratch_types=[
            pltpu.VMEM((G,), jnp.int32),
            pltpu.VMEM((G, C), jnp.float32),
            pltpu.VMEM((QB * C,), jnp.float32),
            pltpu.SemaphoreType.DMA,
        ])
    def body(f_h, nb_h, out_h, idx_v, rows_v, out_v, sem):
        base = _wid() * rows_pw

        def chunk(ci, carry):
            q0 = base + ci * QB
            pltpu.sync_copy(nb_h.at[pl.ds(q0 * KNN, G)], idx_v)
            pltpu.async_copy(f_h.at[idx_v], rows_v, sem).wait()
            for q in range(QB):
                for cc in range(C // 16):
                    acc = rows_v[q * KNN, pl.ds(cc * 16, 16)]
                    for k in range(1, KNN):
                        acc = jnp.maximum(acc, rows_v[q * KNN + k, pl.ds(cc * 16, 16)])
                    out_v[pl.ds(q * C + cc * 16, 16)] = acc
            pltpu.sync_copy(out_v, out_h.at[pl.ds(q0 * C, QB * C)])
            return carry

        lax.fori_loop(0, nchunk, chunk, 0)

    return body(feats, nb_flat).reshape(Mp, C)


def _sc_rowgather(table, idx_flat):
    C = table.shape[1]
    Mp = idx_flat.shape[0]
    QB = 8
    rows_pw = Mp // NW
    nchunk = rows_pw // QB

    @functools.partial(
        pl.kernel,
        out_type=jax.ShapeDtypeStruct((Mp, C), jnp.float32),
        mesh=_sc_mesh(),
        scratch_types=[
            pltpu.VMEM((QB,), jnp.int32),
            pltpu.VMEM((QB, C), jnp.float32),
            pltpu.SemaphoreType.DMA,
        ])
    def body(t_h, i_h, out_h, idx_v, rows_v, sem):
        base = _wid() * rows_pw

        def chunk(ci, carry):
            q0 = base + ci * QB
            pltpu.sync_copy(i_h.at[pl.ds(q0, QB)], idx_v)
            pltpu.async_copy(t_h.at[idx_v], rows_v, sem).wait()
            pltpu.sync_copy(rows_v, out_h.at[pl.ds(q0, QB)])
            return carry

        lax.fori_loop(0, nchunk, chunk, 0)

    return body(table, idx_flat)


def _lanepad16(x):
    return jnp.pad(x, ((0, 0), (0, 16 - x.shape[1])))


def kernel(feats, points0, points1, points2, points3,
           neighbors0, neighbors1, neighbors2, neighbors3,
           subsampling0, subsampling1, subsampling2,
           upsampling1, upsampling2, params):
    p = params
    N1, N2, N3, N4 = points0.shape[0], points1.shape[0], points2.shape[0], points3.shape[0]
    Mp1, Mp2, Mp3, Mp4 = _padM(N1), _padM(N2), _padM(N3), _padM(N4)

    nb0 = _pad_rows(neighbors0, ROW_ALIGN).reshape(-1)
    nb1 = _pad_rows(neighbors1, ROW_ALIGN).reshape(-1)
    nb2 = _pad_rows(neighbors2, ROW_ALIGN).reshape(-1)
    nb3 = _pad_rows(neighbors3, ROW_ALIGN).reshape(-1)
    ss0 = _pad_rows(subsampling0, ROW_ALIGN).reshape(-1)
    ss1 = _pad_rows(subsampling1, ROW_ALIGN).reshape(-1)
    ss2 = _pad_rows(subsampling2, ROW_ALIGN).reshape(-1)

    pl0, pl1, pl2, pl3 = (jnp.pad(x.T, ((0, 0), (0, (-x.shape[0]) % 8)))
                          for x in (points0, points1, points2, points3))
    qp0, qp1, qp2, qp3 = (_pad_rows(_lanepad16(x), ROW_ALIGN)
                          for x in (points0, points1, points2, points3))

    w_n0 = _sc_weights_pl(qp0, pl0, nb0)
    w_s0 = _sc_weights_pl(qp1, pl0, ss0)
    w_n1 = _sc_weights_pl(qp1, pl1, nb1)
    w_s1 = _sc_weights_pl(qp2, pl1, ss1)
    w_n2 = _sc_weights_pl(qp2, pl2, nb2)
    w_s2 = _sc_weights_pl(qp3, pl2, ss2)
    w_n3 = _sc_weights_pl(qp3, pl3, nb3)

    f0 = jnp.pad(feats[:, 0], (0, (-N1) % 8))
    wp1 = _sc_wpool_c1(f0, nb0, w_n0)[:N1]
    wp = jnp.pad(wp1[:, None], ((0, 0), (0, 7)))
    W8 = jnp.pad(p["e11"]["W"], ((0, 7), (0, 0)))
    f1 = _mm(wp, W8, act=True)

    def resnet(f, Mq, wtab, nbf, Mp, pr, strided):
        x = _mm(f, pr["Wa"], act=True)
        xp = _wpool_any(x, nbf, wtab)[:Mq]
        short = _sc_maxpool(f, nbf, Mp)[:Mq] if strided else f
        return _tail(xp, short, pr["Wb"], pr["Wc"], pr.get("Ws"))

    f1 = resnet(f1, N1, w_n0, nb0, Mp1, p["e12"], False)
    f2 = resnet(f1, N2, w_s0, ss0, Mp2, p["e21"], True)
    f2 = resnet(f2, N2, w_n1, nb1, Mp2, p["e22"], False)
    f2 = resnet(f2, N2, w_n1, nb1, Mp2, p["e23"], False)
    f3 = resnet(f2, N3, w_s1, ss1, Mp3, p["e31"], True)
    f3 = resnet(f3, N3, w_n2, nb2, Mp3, p["e32"], False)
    f3 = resnet(f3, N3, w_n2, nb2, Mp3, p["e33"], False)
    f4 = resnet(f3, N4, w_s2, ss2, Mp4, p["e41"], True)
    f4 = resnet(f4, N4, w_n3, nb3, Mp4, p["e42"], False)
    f4 = resnet(f4, N4, w_n3, nb3, Mp4, p["e43"], False)

    up2 = _pad_rows(upsampling2[:, 0], ROW_ALIGN)
    up1 = _pad_rows(upsampling1[:, 0], ROW_ALIGN)
    g3 = _sc_rowgather(f4, up2)[:N3]
    l3 = _dec(g3, f3, p["d3"]["W"], act=True)
    g2 = _sc_rowgather(l3, up1)[:N2]
    l2 = _dec(g2, f2, p["d2"]["W"], act=False)
    return (l2, l3, f4)


# speedup vs baseline: 3.7598x; 1.0017x over previous
import functools
import jax
import jax.numpy as jnp
from jax import lax
from jax.experimental import pallas as pl
from jax.experimental.pallas import tpu as pltpu
from jax.experimental.pallas import tpu_sc as plsc

RADIUS = 0.0625
KNN = 32
BR = 512
NC, NS = 2, 16
NW = NC * NS
ROW_ALIGN = 256

_CP = pltpu.CompilerParams(needs_layout_passes=False)


def _lrelu(x):
    return jnp.where(x >= 0, x, 0.1 * x)


def _pad_rows(x, mult):
    p = (-x.shape[0]) % mult
    if p:
        x = jnp.pad(x, ((0, p),) + ((0, 0),) * (x.ndim - 1))
    return x


def _padM(M):
    return -(-M // ROW_ALIGN) * ROW_ALIGN


def _mm_kernel(x_ref, w_ref, o_ref, *, act):
    y = jnp.dot(x_ref[...], w_ref[...], preferred_element_type=jnp.float32)
    o_ref[...] = _lrelu(y) if act else y


def _mm(x, W, act=True):
    M, Ci = x.shape
    Co = W.shape[1]
    xp = _pad_rows(x, BR)
    Mp = xp.shape[0]
    out = pl.pallas_call(
        functools.partial(_mm_kernel, act=act),
        grid=(Mp // BR,),
        in_specs=[pl.BlockSpec((BR, Ci), lambda i: (i, 0)),
                  pl.BlockSpec((Ci, Co), lambda i: (0, 0))],
        out_specs=pl.BlockSpec((BR, Co), lambda i: (i, 0)),
        out_shape=jax.ShapeDtypeStruct((Mp, Co), jnp.float32),
    )(xp, W)
    return out[:M]


def _tail_kernel(x_ref, s_ref, wb_ref, wc_ref, ws_ref, o_ref):
    h = _lrelu(jnp.dot(x_ref[...], wb_ref[...], preferred_element_type=jnp.float32))
    y = jnp.dot(h, wc_ref[...], preferred_element_type=jnp.float32)
    sh = s_ref[...]
    if ws_ref is not None:
        sh = jnp.dot(sh, ws_ref[...], preferred_element_type=jnp.float32)
    o_ref[...] = _lrelu(y + sh)


def _tail_kernel_nows(x_ref, s_ref, wb_ref, wc_ref, o_ref):
    _tail_kernel(x_ref, s_ref, wb_ref, wc_ref, None, o_ref)


def _tail(x, short, Wb, Wc, Ws=None):
    M, m = x.shape
    o = Wc.shape[1]
    si = short.shape[1]
    xp = _pad_rows(x, BR)
    sp = _pad_rows(short, BR)
    Mp = xp.shape[0]
    specs = [pl.BlockSpec((BR, m), lambda i: (i, 0)),
             pl.BlockSpec((BR, si), lambda i: (i, 0)),
             pl.BlockSpec((m, m), lambda i: (0, 0)),
             pl.BlockSpec((m, o), lambda i: (0, 0))]
    args = [xp, sp, Wb, Wc]
    if Ws is not None:
        specs.append(pl.BlockSpec((si, o), lambda i: (0, 0)))
        args.append(Ws)
        body = _tail_kernel
    else:
        body = _tail_kernel_nows
    out = pl.pallas_call(
        body,
        grid=(Mp // BR,),
        in_specs=specs,
        out_specs=pl.BlockSpec((BR, o), lambda i: (i, 0)),
        out_shape=jax.ShapeDtypeStruct((Mp, o), jnp.float32),
    )(*args)
    return out[:M]


def _dec_kernel(g_ref, f_ref, w1_ref, w2_ref, o_ref, *, act):
    y = jnp.dot(g_ref[...], w1_ref[...], preferred_element_type=jnp.float32)
    y += jnp.dot(f_ref[...], w2_ref[...], preferred_element_type=jnp.float32)
    o_ref[...] = _lrelu(y) if act else y


def _dec(g, f, W, act):
    M, cg = g.shape
    cf = f.shape[1]
    co = W.shape[1]
    W1, W2 = W[:cg], W[cg:]
    gp = _pad_rows(g, BR)
    fp = _pad_rows(f, BR)
    Mp = gp.shape[0]
    out = pl.pallas_call(
        functools.partial(_dec_kernel, act=act),
        grid=(Mp // BR,),
        in_specs=[pl.BlockSpec((BR, cg), lambda i: (i, 0)),
                  pl.BlockSpec((BR, cf), lambda i: (i, 0)),
                  pl.BlockSpec((cg, co), lambda i: (0, 0)),
                  pl.BlockSpec((cf, co), lambda i: (0, 0))],
        out_specs=pl.BlockSpec((BR, co), lambda i: (i, 0)),
        out_shape=jax.ShapeDtypeStruct((Mp, co), jnp.float32),
    )(gp, fp, W1, W2)
    return out[:M]


def _sc_mesh():
    return plsc.VectorSubcoreMesh(core_axis_name="c", subcore_axis_name="s")


def _wid():
    return lax.axis_index("s") * NC + lax.axis_index("c")


def _full(v):
    return jnp.full((16,), v, jnp.int32)


def _sc_weights_pl(qpad, planes, nb_flat):
    Mp = qpad.shape[0]
    qflat = qpad.reshape(-1)
    Ns8 = planes.shape[1]
    QB = 16
    G = QB * KNN
    rows_pw = Mp // NW
    nchunk = rows_pw // QB
    inv_r2 = 1.0 / (RADIUS * RADIUS)

    def make_pass(last):
        nplane = 1 if last else 2

        @functools.partial(
            pl.kernel,
            out_type=jax.ShapeDtypeStruct((Mp * KNN,), jnp.float32),
            mesh=_sc_mesh(),
            compiler_params=_CP,
            scratch_types=[pltpu.VMEM((Ns8,), jnp.float32)] * nplane + [
                pltpu.VMEM((QB * 16,), jnp.float32),
                pltpu.VMEM((G,), jnp.int32),
                pltpu.VMEM((G,), jnp.float32),
                pltpu.VMEM((G,), jnp.float32),
                pltpu.SemaphoreType.DMA,
            ])
        def body(*refs):
            if last:
                (pz_h, qp_h, nb_h, part_h, w_h,
                 pz_v, qp_v, idx_v, part_v, tile_v, sem) = refs
                planes_v = [pz_v]
                pltpu.sync_copy(pz_h, pz_v)
            else:
                (px_h, py_h, qp_h, nb_h, w_h,
                 px_v, py_v, qp_v, idx_v, part_v, tile_v, sem) = refs
                planes_v = [px_v, py_v]
                pltpu.sync_copy(px_h, px_v)
                pltpu.sync_copy(py_h, py_v)
            base = _wid() * rows_pw
            kiota = lax.broadcasted_iota(jnp.int32, (16,), 0)

            def chunk(ci, carry):
                q0 = base + ci * QB
                pltpu.sync_copy(qp_h.at[pl.ds(q0 * 16, QB * 16)], qp_v)
                pltpu.sync_copy(nb_h.at[pl.ds(q0 * KNN, G)], idx_v)
                if last:
                    pltpu.sync_copy(part_h.at[pl.ds(q0 * KNN, G)], part_v)
                    qzv = plsc.load_gather(qp_v, [kiota * 16 + 2])
                    for j in range(KNN):
                        iv = plsc.load_gather(idx_v, [kiota * KNN + j])
                        dz = plsc.load_gather(planes_v[0], [iv]) - qzv
                        pj = plsc.load_gather(part_v, [kiota + j * 16])
                        d2 = pj + dz * dz
                        tile_v[pl.ds(j * 16, 16)] = 1.0 / (1.0 + d2 * inv_r2)
                    for q in range(QB):
                        for g2 in range(KNN // 16):
                            wq = plsc.load_gather(
                                tile_v, [kiota * 16 + g2 * 256 + q])
                            part_v[pl.ds(q * KNN + g2 * 16, 16)] = wq
                    pltpu.sync_copy(part_v, w_h.at[pl.ds(q0 * KNN, G)])
                else:
                    qxv = plsc.load_gather(qp_v, [kiota * 16 + 0])
                    qyv = plsc.load_gather(qp_v, [kiota * 16 + 1])
                    for j in range(KNN):
                        iv = plsc.load_gather(idx_v, [kiota * KNN + j])
                        dx = plsc.load_gather(planes_v[0], [iv]) - qxv
                        dy = plsc.load_gather(planes_v[1], [iv]) - qyv
                        tile_v[pl.ds(j * 16, 16)] = dx * dx + dy * dy
                    pltpu.sync_copy(tile_v, w_h.at[pl.ds(q0 * KNN, G)])
                return carry

            lax.fori_loop(0, nchunk, chunk, 0)

        return body

    part = make_pass(False)(planes[0], planes[1], qflat, nb_flat)
    return make_pass(True)(planes[2], qflat, nb_flat, part)


_WPOOL_QB = {16: 4, 32: 4, 64: 4, 128: 4, 256: 2}


def _sc_wpool(feats, nb_flat, w_flat, C):
    CP = feats.shape[1]
    Mp = w_flat.shape[0] // KNN
    QB = _WPOOL_QB[C]
    G = QB * KNN
    rows_pw = Mp // NW
    nchunk = rows_pw // QB
    scale = 1.0 / KNN

    @functools.partial(
        pl.kernel,
        out_type=jax.ShapeDtypeStruct((Mp * C,), jnp.float32),
        mesh=_sc_mesh(),
        compiler_params=_CP,
        scratch_types=[
            pltpu.VMEM((G,), jnp.int32),
            pltpu.VMEM((G,), jnp.int32),
            pltpu.VMEM((G,), jnp.float32),
            pltpu.VMEM((G,), jnp.float32),
            pltpu.VMEM((G, CP), jnp.float32),
            pltpu.VMEM((G, CP), jnp.float32),
            pltpu.VMEM((QB * C,), jnp.float32),
            pltpu.VMEM((QB * C,), jnp.float32),
            pltpu.SemaphoreType.DMA,
            pltpu.SemaphoreType.DMA,
        ])
    def body(f_h, nb_h, w_h, out_h, idxA, idxB, wA, wB, rowsA, rowsB,
             outA, outB, semA, semB):
        base = _wid() * rows_pw
        kiota = lax.broadcasted_iota(jnp.int32, (16,), 0)
        bufs = [(idxA, wA, rowsA, outA, semA), (idxB, wB, rowsB, outB, semB)]

        def compute(w_v, rows_v, out_v):
            for q in range(QB):
                accs = [jnp.zeros((16,), jnp.float32) for _ in range(C // 16)]
                for k in range(KNN):
                    wk = plsc.load_gather(w_v, [_full(q * KNN + k)])
                    row = q * KNN + k
                    for cc in range(C // 16):
                        rv = plsc.load_gather(rows_v, [_full(row), kiota + cc * 16])
                        accs[cc] = accs[cc] + wk * rv
                for cc in range(C // 16):
                    out_v[pl.ds(q * C + cc * 16, 16)] = accs[cc] * scale

        def pair(pi, carry):
            dmas = []
            for g in range(2):
                q0 = base + (2 * pi + g) * QB
                idx_v, w_v, rows_v, out_v, sem = bufs[g]
                pltpu.sync_copy(nb_h.at[pl.ds(q0 * KNN, G)], idx_v)
                pltpu.sync_copy(w_h.at[pl.ds(q0 * KNN, G)], w_v)
                dmas.append(pltpu.async_copy(f_h.at[idx_v], rows_v, sem))
            for g in range(2):
                q0 = base + (2 * pi + g) * QB
                idx_v, w_v, rows_v, out_v, sem = bufs[g]
                dmas[g].wait()
                compute(w_v, rows_v, out_v)
                pltpu.sync_copy(out_v, out_h.at[pl.ds(q0 * C, QB * C)])
            return carry

        lax.fori_loop(0, nchunk // 2, pair, 0)

    return body(feats, nb_flat, w_flat).reshape(Mp, C)


def _sc_wpool_c1(fflat, nb_flat, w_flat):
    Nf = fflat.shape[0]
    Mp = w_flat.shape[0] // KNN
    QB = 16
    G = QB * KNN
    rows_pw = Mp // NW
    nchunk = rows_pw // QB
    scale = 1.0 / KNN

    @functools.partial(
        pl.kernel,
        out_type=jax.ShapeDtypeStruct((Mp,), jnp.float32),
        mesh=_sc_mesh(),
        compiler_params=_CP,
        scratch_types=[
            pltpu.VMEM((Nf,), jnp.float32),
            pltpu.VMEM((G,), jnp.int32),
            pltpu.VMEM((G,), jnp.float32),
            pltpu.VMEM((QB,), jnp.float32),
            pltpu.SemaphoreType.DMA,
        ])
    def body(f_h, nb_h, w_h, out_h, f_v, idx_v, w_v, out_v, sem):
        pltpu.sync_copy(f_h, f_v)
        base = _wid() * rows_pw
        kiota = lax.broadcasted_iota(jnp.int32, (16,), 0)

        def chunk(ci, carry):
            q0 = base + ci * QB
            pltpu.sync_copy(nb_h.at[pl.ds(q0 * KNN, G)], idx_v)
            pltpu.sync_copy(w_h.at[pl.ds(q0 * KNN, G)], w_v)
            acc = jnp.zeros((16,), jnp.float32)
            for j in range(KNN):
                iv = plsc.load_gather(idx_v, [kiota * KNN + j])
                fv = plsc.load_gather(f_v, [iv])
                wv = plsc.load_gather(w_v, [kiota * KNN + j])
                acc = acc + fv * wv
            out_v[...] = acc * scale
            pltpu.sync_copy(out_v, out_h.at[pl.ds(q0, QB)])
            return carry

        lax.fori_loop(0, nchunk, chunk, 0)

    return body(fflat, nb_flat, w_flat)


def _wpool_any(feats, nb_flat, w_flat):
    Ns, C = feats.shape
    if C < 128:
        feats = jnp.pad(feats, ((0, 0), (0, 128 - C)))
    return _sc_wpool(feats, nb_flat, w_flat, C)


_MAXP_QB = {128: 4, 256: 2, 512: 1}


def _sc_maxpool(feats, nb_flat, Mp):
    C = feats.shape[1]
    QB = _MAXP_QB[C]
    G = QB * KNN
    rows_pw = Mp // NW
    nchunk = rows_pw // QB

    @functools.partial(
        pl.kernel,
        out_type=jax.ShapeDtypeStruct((Mp * C,), jnp.float32),
        mesh=_sc_mesh(),
        scratch_types=[
            pltpu.VMEM((G,), jnp.int32),
            pltpu.VMEM((G, C), jnp.float32),
            pltpu.VMEM((QB * C,), jnp.float32),
            pltpu.SemaphoreType.DMA,
        ])
    def body(f_h, nb_h, out_h, idx_v, rows_v, out_v, sem):
        base = _wid() * rows_pw

        def chunk(ci, carry):
            q0 = base + ci * QB
            pltpu.sync_copy(nb_h.at[pl.ds(q0 * KNN, G)], idx_v)
            pltpu.async_copy(f_h.at[idx_v], rows_v, sem).wait()
            for q in range(QB):
                for cc in range(C // 16):
                    acc = rows_v[q * KNN, pl.ds(cc * 16, 16)]
                    for k in range(1, KNN):
                        acc = jnp.maximum(acc, rows_v[q * KNN + k, pl.ds(cc * 16, 16)])
                    out_v[pl.ds(q * C + cc * 16, 16)] = acc
            pltpu.sync_copy(out_v, out_h.at[pl.ds(q0 * C, QB * C)])
            return carry

        lax.fori_loop(0, nchunk, chunk, 0)

    return body(feats, nb_flat).reshape(Mp, C)


def _sc_rowgather(table, idx_flat):
    C = table.shape[1]
    Mp = idx_flat.shape[0]
    QB = 8
    rows_pw = Mp // NW
    nchunk = rows_pw // QB

    @functools.partial(
        pl.kernel,
        out_type=jax.ShapeDtypeStruct((Mp, C), jnp.float32),
        mesh=_sc_mesh(),
        scratch_types=[
            pltpu.VMEM((QB,), jnp.int32),
            pltpu.VMEM((QB, C), jnp.float32),
            pltpu.SemaphoreType.DMA,
        ])
    def body(t_h, i_h, out_h, idx_v, rows_v, sem):
        base = _wid() * rows_pw

        def chunk(ci, carry):
            q0 = base + ci * QB
            pltpu.sync_copy(i_h.at[pl.ds(q0, QB)], idx_v)
            pltpu.async_copy(t_h.at[idx_v], rows_v, sem).wait()
            pltpu.sync_copy(rows_v, out_h.at[pl.ds(q0, QB)])
            return carry

        lax.fori_loop(0, nchunk, chunk, 0)

    return body(table, idx_flat)


def _lanepad16(x):
    return jnp.pad(x, ((0, 0), (0, 16 - x.shape[1])))


def kernel(feats, points0, points1, points2, points3,
           neighbors0, neighbors1, neighbors2, neighbors3,
           subsampling0, subsampling1, subsampling2,
           upsampling1, upsampling2, params):
    p = params
    N1, N2, N3, N4 = points0.shape[0], points1.shape[0], points2.shape[0], points3.shape[0]
    Mp1, Mp2, Mp3, Mp4 = _padM(N1), _padM(N2), _padM(N3), _padM(N4)

    nb0 = _pad_rows(neighbors0, ROW_ALIGN).reshape(-1)
    nb1 = _pad_rows(neighbors1, ROW_ALIGN).reshape(-1)
    nb2 = _pad_rows(neighbors2, ROW_ALIGN).reshape(-1)
    nb3 = _pad_rows(neighbors3, ROW_ALIGN).reshape(-1)
    ss0 = _pad_rows(subsampling0, ROW_ALIGN).reshape(-1)
    ss1 = _pad_rows(subsampling1, ROW_ALIGN).reshape(-1)
    ss2 = _pad_rows(subsampling2, ROW_ALIGN).reshape(-1)

    pl0, pl1, pl2, pl3 = (jnp.pad(x.T, ((0, 0), (0, (-x.shape[0]) % 8)))
                          for x in (points0, points1, points2, points3))
    qp0, qp1, qp2, qp3 = (_pad_rows(_lanepad16(x), ROW_ALIGN)
                          for x in (points0, points1, points2, points3))

    w_n0 = _sc_weights_pl(qp0, pl0, nb0)
    w_s0 = _sc_weights_pl(qp1, pl0, ss0)
    w_n1 = _sc_weights_pl(qp1, pl1, nb1)
    w_s1 = _sc_weights_pl(qp2, pl1, ss1)
    w_n2 = _sc_weights_pl(qp2, pl2, nb2)
    w_s2 = _sc_weights_pl(qp3, pl2, ss2)
    w_n3 = _sc_weights_pl(qp3, pl3, nb3)

    f0 = jnp.pad(feats[:, 0], (0, (-N1) % 8))
    wp1 = _sc_wpool_c1(f0, nb0, w_n0)[:N1]
    wp = jnp.pad(wp1[:, None], ((0, 0), (0, 7)))
    W8 = jnp.pad(p["e11"]["W"], ((0, 7), (0, 0)))
    f1 = _mm(wp, W8, act=True)

    def resnet(f, Mq, wtab, nbf, Mp, pr, strided):
        x = _mm(f, pr["Wa"], act=True)
        xp = _wpool_any(x, nbf, wtab)[:Mq]
        short = _sc_maxpool(f, nbf, Mp)[:Mq] if strided else f
        return _tail(xp, short, pr["Wb"], pr["Wc"], pr.get("Ws"))

    f1 = resnet(f1, N1, w_n0, nb0, Mp1, p["e12"], False)
    f2 = resnet(f1, N2, w_s0, ss0, Mp2, p["e21"], True)
    f2 = resnet(f2, N2, w_n1, nb1, Mp2, p["e22"], False)
    f2 = resnet(f2, N2, w_n1, nb1, Mp2, p["e23"], False)
    f3 = resnet(f2, N3, w_s1, ss1, Mp3, p["e31"], True)
    f3 = resnet(f3, N3, w_n2, nb2, Mp3, p["e32"], False)
    f3 = resnet(f3, N3, w_n2, nb2, Mp3, p["e33"], False)
    f4 = resnet(f3, N4, w_s2, ss2, Mp4, p["e41"], True)
    f4 = resnet(f4, N4, w_n3, nb3, Mp4, p["e42"], False)
    f4 = resnet(f4, N4, w_n3, nb3, Mp4, p["e43"], False)

    up2 = _pad_rows(upsampling2[:, 0], ROW_ALIGN)
    up1 = _pad_rows(upsampling1[:, 0], ROW_ALIGN)
    g3 = _sc_rowgather(f4, up2)[:N3]
    l3 = _dec(g3, f3, p["d3"]["W"], act=True)
    g2 = _sc_rowgather(l3, up1)[:N2]
    l2 = _dec(g2, f2, p["d2"]["W"], act=False)
    return (l2, l3, f4)
